# Initial kernel scaffold; baseline (speedup 1.0000x reference)
import jax
import jax.numpy as jnp
from jax.experimental import pallas as pl


def kernel(x, edge_index, edge_attr, x_pos, W_a1, b_a1, Wn1, bn1, Wn2, bn2, Wn3, bn3, W_a2, b_a2, W_a3, b_a3):
    raise NotImplementedError("write your pallas kernel here")


# speedup vs baseline: 2.9597x; 2.9597x over previous
import functools

import jax
import jax.numpy as jnp
import numpy as np
from jax import lax
from jax.experimental import pallas as pl
from jax.experimental.pallas import tpu as pltpu
from jax.experimental.pallas import tpu_sc as plsc

NC = 2
NS = 16
NW = NC * NS
CH = 80


def _gelu(u):
    return 0.5 * u * (1.0 + lax.erf(u * np.float32(1.0 / np.sqrt(2.0))))


def _node_stage(x, W_a1, b_a1, Wg):
    N, C = x.shape
    BN = 1000
    grid = (N // BN,)

    def body(x_ref, wa1_ref, ba1_ref, wg_ref, h_ref, t_ref):
        hb = x_ref[...] @ wa1_ref[...] + ba1_ref[...]
        h_ref[...] = hb
        g = hb @ wg_ref[...]
        t_ref[...] = jnp.concatenate([g, jnp.zeros_like(g)], axis=1)

    return pl.pallas_call(
        body,
        grid=grid,
        in_specs=[
            pl.BlockSpec((BN, C), lambda i: (i, 0)),
            pl.BlockSpec((C, C), lambda i: (0, 0)),
            pl.BlockSpec((1, C), lambda i: (0, 0)),
            pl.BlockSpec((C, 64), lambda i: (0, 0)),
        ],
        out_specs=[
            pl.BlockSpec((BN, C), lambda i: (i, 0)),
            pl.BlockSpec((BN, C), lambda i: (i, 0)),
        ],
        out_shape=[
            jax.ShapeDtypeStruct((N, C), jnp.float32),
            jax.ShapeDtypeStruct((N, C), jnp.float32),
        ],
    )(x, W_a1, b_a1, Wg)


def _sc_gather(src, dst, T, px, py, pz):
    E = src.shape[0]
    N, C = T.shape
    EPW = E // NW
    NCHK = EPW // CH
    mesh = plsc.VectorSubcoreMesh(
        core_axis_name="c", subcore_axis_name="s", num_cores=NC, num_subcores=NS
    )

    @functools.partial(
        pl.kernel,
        mesh=mesh,
        out_type=jax.ShapeDtypeStruct((E, C), jnp.float32),
        compiler_params=pltpu.CompilerParams(needs_layout_passes=False),
        scratch_types=[
            pltpu.VMEM((EPW,), jnp.int32),
            pltpu.VMEM((EPW,), jnp.int32),
            pltpu.VMEM((N,), jnp.float32),
            pltpu.VMEM((N,), jnp.float32),
            pltpu.VMEM((N,), jnp.float32),
            pltpu.VMEM((CH, C), jnp.float32),
            pltpu.SemaphoreType.DMA,
        ],
    )
    def k(src_hbm, dst_hbm, t_hbm, px_hbm, py_hbm, pz_hbm, tj_out,
          src_v, dst_v, px_v, py_v, pz_v, tjbuf, semt):
        w = lax.axis_index("c") * NS + lax.axis_index("s")
        base = pl.multiple_of(w * EPW, 8)
        pltpu.sync_copy(src_hbm.at[pl.ds(base, EPW)], src_v)
        pltpu.sync_copy(dst_hbm.at[pl.ds(base, EPW)], dst_v)
        pltpu.sync_copy(px_hbm, px_v)
        pltpu.sync_copy(py_hbm, py_v)
        pltpu.sync_copy(pz_hbm, pz_v)
        col64 = jnp.full((16,), 64, jnp.int32)
        lane = lax.iota(jnp.int32, 16)

        @pl.loop(0, NCHK)
        def _(j):
            ch0 = pl.multiple_of(j * CH, 8)
            ct = pltpu.async_copy(
                t_hbm.at[src_v.at[pl.ds(ch0, CH)]], tjbuf, semt)
            ct.wait()
            for gi in range(CH // 16):
                off = pl.multiple_of(j * CH + gi * 16, 8)
                s16 = src_v[pl.ds(off, 16)]
                d16 = dst_v[pl.ds(off, 16)]
                dx = plsc.load_gather(px_v, [d16]) - plsc.load_gather(px_v, [s16])
                dy = plsc.load_gather(py_v, [d16]) - plsc.load_gather(py_v, [s16])
                dz = plsc.load_gather(pz_v, [d16]) - plsc.load_gather(pz_v, [s16])
                r2 = dx * dx + dy * dy + dz * dz
                plsc.store_scatter(tjbuf, [gi * 16 + lane, col64], r2)
            pltpu.sync_copy(
                tjbuf, tj_out.at[pl.ds(pl.multiple_of(base + j * CH, 8), CH)])

    return k(src, dst, T, px, py, pz)


def _edge_stage(TJ, edge_attr, Wsc, We, bn1, Wn2, bn2, Wn3, bn3, omeg):
    E, C = edge_attr.shape
    MID = Wn2.shape[0]
    BE = 2560
    grid = (E // BE,)

    def body(tj_ref, ea_ref, wsc_ref, we_ref, bn1_ref, w2_ref,
             bn2_ref, w3_ref, bn3_ref, om_ref, out_ref):
        tj = tj_ref[...]
        g = tj[:, :64]
        r = jnp.sqrt(tj[:, 64:65])
        ang = r * om_ref[...]
        sc = jnp.concatenate([jnp.sin(ang), jnp.cos(ang)], axis=1)
        pre = sc @ wsc_ref[...] + ea_ref[...] @ we_ref[...] + g + bn1_ref[...]
        h1 = _gelu(pre)
        h2 = _gelu(h1 @ w2_ref[...] + bn2_ref[...])
        out_ref[...] = h2 @ w3_ref[...] + bn3_ref[...]

    return pl.pallas_call(
        body,
        grid=grid,
        in_specs=[
            pl.BlockSpec((BE, C), lambda i: (i, 0)),
            pl.BlockSpec((BE, C), lambda i: (i, 0)),
            pl.BlockSpec((32, MID), lambda i: (0, 0)),
            pl.BlockSpec((C, MID), lambda i: (0, 0)),
            pl.BlockSpec((1, MID), lambda i: (0, 0)),
            pl.BlockSpec((MID, MID), lambda i: (0, 0)),
            pl.BlockSpec((1, MID), lambda i: (0, 0)),
            pl.BlockSpec((MID, C), lambda i: (0, 0)),
            pl.BlockSpec((1, C), lambda i: (0, 0)),
            pl.BlockSpec((1, 16), lambda i: (0, 0)),
        ],
        out_specs=pl.BlockSpec((BE, C), lambda i: (i, 0)),
        out_shape=jax.ShapeDtypeStruct((E, C), jnp.float32),
    )(TJ, edge_attr, Wsc, We, bn1, Wn2, bn2, Wn3, bn3, omeg)


def _sc_scatter(dst3d, msg, zeros):
    E, C = msg.shape
    N = zeros.shape[0]
    EPW = E // NW
    NCHK = EPW // CH
    RPT = 640
    TAIL = N - RPT * (NS - 1)
    mesh = plsc.VectorSubcoreMesh(
        core_axis_name="c", subcore_axis_name="s", num_cores=NC, num_subcores=NS
    )

    @functools.partial(
        pl.kernel,
        mesh=mesh,
        compiler_params=pltpu.CompilerParams(needs_layout_passes=False),
        out_type=jax.ShapeDtypeStruct((NC * N, C), jnp.float32),
        scratch_types=[
            pltpu.VMEM((NCHK, CH), jnp.int32),
            pltpu.VMEM((CH, C), jnp.float32),
            pltpu.VMEM_SHARED((N, C), jnp.float32),
        ],
    )
    def k(dst3d_hbm, msg_hbm, z_hbm, agg_out, idx_v, msgbuf, agg_sh):
        c = lax.axis_index("c")
        s = lax.axis_index("s")
        w = c * NS + s

        @pl.when(s < NS - 1)
        def _():
            r0 = pl.multiple_of(s * RPT, 8)
            pltpu.sync_copy(z_hbm.at[pl.ds(r0, RPT)], agg_sh.at[pl.ds(r0, RPT)])

        @pl.when(s == NS - 1)
        def _():
            r0 = RPT * (NS - 1)
            pltpu.sync_copy(z_hbm.at[pl.ds(r0, TAIL)], agg_sh.at[pl.ds(r0, TAIL)])

        pltpu.sync_copy(dst3d_hbm.at[w], idx_v)
        plsc.subcore_barrier()

        @pl.loop(0, NCHK)
        def _(j):
            e0 = pl.multiple_of(w * EPW + j * CH, 8)
            pltpu.sync_copy(msg_hbm.at[pl.ds(e0, CH)], msgbuf)
            pltpu.sync_copy(msgbuf, agg_sh.at[idx_v.at[j]], add=True)

        plsc.subcore_barrier()

        @pl.when(s < NS - 1)
        def _():
            r0 = pl.multiple_of(s * RPT, 8)
            o0 = pl.multiple_of(c * N + s * RPT, 8)
            pltpu.sync_copy(agg_sh.at[pl.ds(r0, RPT)], agg_out.at[pl.ds(o0, RPT)])

        @pl.when(s == NS - 1)
        def _():
            r0 = RPT * (NS - 1)
            o0 = pl.multiple_of(c * N + r0, 8)
            pltpu.sync_copy(agg_sh.at[pl.ds(r0, TAIL)], agg_out.at[pl.ds(o0, TAIL)])

    return k(dst3d, msg, zeros)


def _final_stage(h, aggp, W_a2, b_a2, W_a3, b_a3):
    N, C = h.shape
    BN = 1000
    grid = (N // BN,)

    def body(h_ref, a0_ref, a1_ref, wa2_ref, ba2_ref, wa3_ref, ba3_ref, out_ref):
        hb = h_ref[...] + a0_ref[...] + a1_ref[...]
        t = jnp.maximum(hb @ wa2_ref[...] + ba2_ref[...], 0.0)
        out_ref[...] = t @ wa3_ref[...] + ba3_ref[...]

    nb = N // BN
    return pl.pallas_call(
        body,
        grid=grid,
        in_specs=[
            pl.BlockSpec((BN, C), lambda i: (i, 0)),
            pl.BlockSpec((BN, C), lambda i: (i, 0)),
            pl.BlockSpec((BN, C), lambda i: (i + nb, 0)),
            pl.BlockSpec((C, C), lambda i: (0, 0)),
            pl.BlockSpec((1, C), lambda i: (0, 0)),
            pl.BlockSpec((C, C), lambda i: (0, 0)),
            pl.BlockSpec((1, C), lambda i: (0, 0)),
        ],
        out_specs=pl.BlockSpec((BN, C), lambda i: (i, 0)),
        out_shape=jax.ShapeDtypeStruct((N, C), jnp.float32),
    )(h, aggp, aggp, W_a2, b_a2, W_a3, b_a3)


def kernel(x, edge_index, edge_attr, x_pos, W_a1, b_a1, Wn1, bn1, Wn2, bn2,
           Wn3, bn3, W_a2, b_a2, W_a3, b_a3):
    N, C = x.shape
    E = edge_index.shape[1]
    NF = 16
    n_channels = 128
    omeg = jnp.asarray(
        [10.0 * (float(n_channels) ** (1.0 - 2.0 * i / NF)) for i in range(NF)],
        jnp.float32).reshape(1, NF)

    src = edge_index[0]
    dst = edge_index[1]
    Wsc = Wn1[: 2 * NF]
    We = Wn1[2 * NF: 2 * NF + C]
    Wg = Wn1[2 * NF + C:]

    h, T = _node_stage(x, W_a1, b_a1.reshape(1, C), Wg)
    TJ = _sc_gather(src, dst, T, x_pos[:, 0], x_pos[:, 1], x_pos[:, 2])
    msg = _edge_stage(TJ, edge_attr, Wsc, We, bn1.reshape(1, -1), Wn2,
                      bn2.reshape(1, -1), Wn3, bn3.reshape(1, C), omeg)
    aggp = _sc_scatter(dst.reshape(NW, E // (NW * CH), CH), msg,
                       jnp.zeros((N, C), jnp.float32))
    return _final_stage(h, aggp, W_a2, b_a2.reshape(1, C), W_a3,
                        b_a3.reshape(1, C))


# speedup vs baseline: 4.1336x; 1.3966x over previous
import functools

import jax
import jax.numpy as jnp
import numpy as np
from jax import lax
from jax.experimental import pallas as pl
from jax.experimental.pallas import tpu as pltpu
from jax.experimental.pallas import tpu_sc as plsc

NC = 2
NS = 16
NW = NC * NS
CH = 80


def _gelu(u):
    return 0.5 * u * (1.0 + lax.erf(u * np.float32(1.0 / np.sqrt(2.0))))


def _sincos(ang):
    nf = jnp.round(ang * np.float32(2.0 / np.pi))
    ni = nf.astype(jnp.int32)
    x = ((ang - nf * np.float32(1.5703125))
         - nf * np.float32(4.837512969970703125e-4)) \
        - nf * np.float32(7.54978995489188216e-8)
    z = x * x
    sp = ((np.float32(-1.9515295891e-4) * z + np.float32(8.3321608736e-3)) * z
          + np.float32(-1.6666654611e-1)) * z * x + x
    cp = ((np.float32(2.443315711809948e-5) * z
           + np.float32(-1.388731625493765e-3)) * z
          + np.float32(4.166664568298827e-2)) * z * z \
        - np.float32(0.5) * z + np.float32(1.0)
    q0 = (ni & 1) != 0
    q1 = (ni & 2) != 0
    s = jnp.where(q0, cp, sp)
    s = jnp.where(q1, -s, s)
    c = jnp.where(q0, sp, cp)
    c = jnp.where(((ni + 1) & 2) != 0, -c, c)
    return s, c


def _node_stage(x, W_a1, b_a1, Wg):
    N, C = x.shape
    BN = 1000
    grid = (N // BN,)

    def body(x_ref, wa1_ref, ba1_ref, wg_ref, h_ref, t_ref):
        hb = x_ref[...] @ wa1_ref[...] + ba1_ref[...]
        h_ref[...] = hb
        g = hb @ wg_ref[...]
        t_ref[...] = jnp.concatenate([g, jnp.zeros_like(g)], axis=1)

    return pl.pallas_call(
        body,
        grid=grid,
        in_specs=[
            pl.BlockSpec((BN, C), lambda i: (i, 0)),
            pl.BlockSpec((C, C), lambda i: (0, 0)),
            pl.BlockSpec((1, C), lambda i: (0, 0)),
            pl.BlockSpec((C, 64), lambda i: (0, 0)),
        ],
        out_specs=[
            pl.BlockSpec((BN, C), lambda i: (i, 0)),
            pl.BlockSpec((BN, C), lambda i: (i, 0)),
        ],
        out_shape=[
            jax.ShapeDtypeStruct((N, C), jnp.float32),
            jax.ShapeDtypeStruct((N, C), jnp.float32),
        ],
    )(x, W_a1, b_a1, Wg)


def _sc_gather(src, dst, T, px, py, pz):
    E = src.shape[0]
    N, C = T.shape
    EPW = E // NW
    NCHK = EPW // CH
    mesh = plsc.VectorSubcoreMesh(
        core_axis_name="c", subcore_axis_name="s", num_cores=NC, num_subcores=NS
    )

    @functools.partial(
        pl.kernel,
        mesh=mesh,
        out_type=jax.ShapeDtypeStruct((E, C), jnp.float32),
        compiler_params=pltpu.CompilerParams(needs_layout_passes=False),
        scratch_types=[
            pltpu.VMEM((EPW,), jnp.int32),
            pltpu.VMEM((EPW,), jnp.int32),
            pltpu.VMEM((N,), jnp.float32),
            pltpu.VMEM((N,), jnp.float32),
            pltpu.VMEM((N,), jnp.float32),
            pltpu.VMEM((CH, C), jnp.float32),
            pltpu.SemaphoreType.DMA,
        ],
    )
    def k(src_hbm, dst_hbm, t_hbm, px_hbm, py_hbm, pz_hbm, tj_out,
          src_v, dst_v, px_v, py_v, pz_v, tjbuf, semt):
        w = lax.axis_index("c") * NS + lax.axis_index("s")
        base = pl.multiple_of(w * EPW, 8)
        pltpu.sync_copy(src_hbm.at[pl.ds(base, EPW)], src_v)
        pltpu.sync_copy(dst_hbm.at[pl.ds(base, EPW)], dst_v)
        pltpu.sync_copy(px_hbm, px_v)
        pltpu.sync_copy(py_hbm, py_v)
        pltpu.sync_copy(pz_hbm, pz_v)
        col64 = jnp.full((16,), 64, jnp.int32)
        lane = lax.iota(jnp.int32, 16)

        @pl.loop(0, NCHK)
        def _(j):
            ch0 = pl.multiple_of(j * CH, 8)
            ct = pltpu.async_copy(
                t_hbm.at[src_v.at[pl.ds(ch0, CH)]], tjbuf, semt)
            ct.wait()
            for gi in range(CH // 16):
                off = pl.multiple_of(j * CH + gi * 16, 8)
                s16 = src_v[pl.ds(off, 16)]
                d16 = dst_v[pl.ds(off, 16)]
                dx = plsc.load_gather(px_v, [d16]) - plsc.load_gather(px_v, [s16])
                dy = plsc.load_gather(py_v, [d16]) - plsc.load_gather(py_v, [s16])
                dz = plsc.load_gather(pz_v, [d16]) - plsc.load_gather(pz_v, [s16])
                r2 = dx * dx + dy * dy + dz * dz
                plsc.store_scatter(tjbuf, [gi * 16 + lane, col64], r2)
            pltpu.sync_copy(
                tjbuf, tj_out.at[pl.ds(pl.multiple_of(base + j * CH, 8), CH)])

    return k(src, dst, T, px, py, pz)


def _edge_stage(TJ, edge_attr, Wsc, We, bn1, Wn2, bn2, Wn3, bn3, omeg):
    E, C = edge_attr.shape
    MID = Wn2.shape[0]
    BE = 2560
    grid = (E // BE,)

    def body(tj_ref, ea_ref, wsc_ref, we_ref, bn1_ref, w2_ref,
             bn2_ref, w3_ref, bn3_ref, om_ref, out_ref):
        tj = tj_ref[...]
        g = tj[:, :64]
        r = jnp.sqrt(tj[:, 64:65])
        ang = r * om_ref[...]
        sn, cs = _sincos(ang)
        sc = jnp.concatenate([sn, cs], axis=1)
        pre = sc @ wsc_ref[...] + ea_ref[...] @ we_ref[...] + g + bn1_ref[...]
        h1 = _gelu(pre)
        h2 = _gelu(h1 @ w2_ref[...] + bn2_ref[...])
        out_ref[...] = h2 @ w3_ref[...] + bn3_ref[...]

    return pl.pallas_call(
        body,
        grid=grid,
        in_specs=[
            pl.BlockSpec((BE, C), lambda i: (i, 0)),
            pl.BlockSpec((BE, C), lambda i: (i, 0)),
            pl.BlockSpec((32, MID), lambda i: (0, 0)),
            pl.BlockSpec((C, MID), lambda i: (0, 0)),
            pl.BlockSpec((1, MID), lambda i: (0, 0)),
            pl.BlockSpec((MID, MID), lambda i: (0, 0)),
            pl.BlockSpec((1, MID), lambda i: (0, 0)),
            pl.BlockSpec((MID, C), lambda i: (0, 0)),
            pl.BlockSpec((1, C), lambda i: (0, 0)),
            pl.BlockSpec((1, 16), lambda i: (0, 0)),
        ],
        out_specs=pl.BlockSpec((BE, C), lambda i: (i, 0)),
        out_shape=jax.ShapeDtypeStruct((E, C), jnp.float32),
    )(TJ, edge_attr, Wsc, We, bn1, Wn2, bn2, Wn3, bn3, omeg)


def _sc_scatter(dst3d, msg, zeros):
    E, C = msg.shape
    N = zeros.shape[0]
    EPW = E // NW
    NCHK = EPW // CH
    RPT = 640
    TAIL = N - RPT * (NS - 1)
    mesh = plsc.VectorSubcoreMesh(
        core_axis_name="c", subcore_axis_name="s", num_cores=NC, num_subcores=NS
    )

    @functools.partial(
        pl.kernel,
        mesh=mesh,
        compiler_params=pltpu.CompilerParams(needs_layout_passes=False),
        out_type=jax.ShapeDtypeStruct((NC * N, C), jnp.float32),
        scratch_types=[
            pltpu.VMEM((NCHK, CH), jnp.int32),
            pltpu.VMEM((CH, C), jnp.float32),
            pltpu.VMEM_SHARED((N, C), jnp.float32),
        ],
    )
    def k(dst3d_hbm, msg_hbm, z_hbm, agg_out, idx_v, msgbuf, agg_sh):
        c = lax.axis_index("c")
        s = lax.axis_index("s")
        w = c * NS + s

        @pl.when(s < NS - 1)
        def _():
            r0 = pl.multiple_of(s * RPT, 8)
            pltpu.sync_copy(z_hbm.at[pl.ds(r0, RPT)], agg_sh.at[pl.ds(r0, RPT)])

        @pl.when(s == NS - 1)
        def _():
            r0 = RPT * (NS - 1)
            pltpu.sync_copy(z_hbm.at[pl.ds(r0, TAIL)], agg_sh.at[pl.ds(r0, TAIL)])

        pltpu.sync_copy(dst3d_hbm.at[w], idx_v)
        plsc.subcore_barrier()

        @pl.loop(0, NCHK)
        def _(j):
            e0 = pl.multiple_of(w * EPW + j * CH, 8)
            pltpu.sync_copy(msg_hbm.at[pl.ds(e0, CH)], msgbuf)
            pltpu.sync_copy(msgbuf, agg_sh.at[idx_v.at[j]], add=True)

        plsc.subcore_barrier()

        @pl.when(s < NS - 1)
        def _():
            r0 = pl.multiple_of(s * RPT, 8)
            o0 = pl.multiple_of(c * N + s * RPT, 8)
            pltpu.sync_copy(agg_sh.at[pl.ds(r0, RPT)], agg_out.at[pl.ds(o0, RPT)])

        @pl.when(s == NS - 1)
        def _():
            r0 = RPT * (NS - 1)
            o0 = pl.multiple_of(c * N + r0, 8)
            pltpu.sync_copy(agg_sh.at[pl.ds(r0, TAIL)], agg_out.at[pl.ds(o0, TAIL)])

    return k(dst3d, msg, zeros)


def _final_stage(h, aggp, W_a2, b_a2, W_a3, b_a3):
    N, C = h.shape
    BN = 1000
    grid = (N // BN,)

    def body(h_ref, a0_ref, a1_ref, wa2_ref, ba2_ref, wa3_ref, ba3_ref, out_ref):
        hb = h_ref[...] + a0_ref[...] + a1_ref[...]
        t = jnp.maximum(hb @ wa2_ref[...] + ba2_ref[...], 0.0)
        out_ref[...] = t @ wa3_ref[...] + ba3_ref[...]

    nb = N // BN
    return pl.pallas_call(
        body,
        grid=grid,
        in_specs=[
            pl.BlockSpec((BN, C), lambda i: (i, 0)),
            pl.BlockSpec((BN, C), lambda i: (i, 0)),
            pl.BlockSpec((BN, C), lambda i: (i + nb, 0)),
            pl.BlockSpec((C, C), lambda i: (0, 0)),
            pl.BlockSpec((1, C), lambda i: (0, 0)),
            pl.BlockSpec((C, C), lambda i: (0, 0)),
            pl.BlockSpec((1, C), lambda i: (0, 0)),
        ],
        out_specs=pl.BlockSpec((BN, C), lambda i: (i, 0)),
        out_shape=jax.ShapeDtypeStruct((N, C), jnp.float32),
    )(h, aggp, aggp, W_a2, b_a2, W_a3, b_a3)


def kernel(x, edge_index, edge_attr, x_pos, W_a1, b_a1, Wn1, bn1, Wn2, bn2,
           Wn3, bn3, W_a2, b_a2, W_a3, b_a3):
    N, C = x.shape
    E = edge_index.shape[1]
    NF = 16
    n_channels = 128
    omeg = jnp.asarray(
        [10.0 * (float(n_channels) ** (1.0 - 2.0 * i / NF)) for i in range(NF)],
        jnp.float32).reshape(1, NF)

    src = edge_index[0]
    dst = edge_index[1]
    Wsc = Wn1[: 2 * NF]
    We = Wn1[2 * NF: 2 * NF + C]
    Wg = Wn1[2 * NF + C:]

    h, T = _node_stage(x, W_a1, b_a1.reshape(1, C), Wg)
    TJ = _sc_gather(src, dst, T, x_pos[:, 0], x_pos[:, 1], x_pos[:, 2])
    msg = _edge_stage(TJ, edge_attr, Wsc, We, bn1.reshape(1, -1), Wn2,
                      bn2.reshape(1, -1), Wn3, bn3.reshape(1, C), omeg)
    aggp = _sc_scatter(dst.reshape(NW, E // (NW * CH), CH), msg,
                       jnp.zeros((N, C), jnp.float32))
    return _final_stage(h, aggp, W_a2, b_a2.reshape(1, C), W_a3,
                        b_a3.reshape(1, C))


# speedup vs baseline: 4.9931x; 1.2079x over previous
import functools

import jax
import jax.numpy as jnp
import numpy as np
from jax import lax
from jax.experimental import pallas as pl
from jax.experimental.pallas import tpu as pltpu
from jax.experimental.pallas import tpu_sc as plsc

NC = 2
NS = 16
NW = NC * NS
CH = 80


def _gelu(u):
    return 0.5 * u * (1.0 + lax.erf(u * np.float32(1.0 / np.sqrt(2.0))))


def _sincos(ang):
    nf = jnp.round(ang * np.float32(2.0 / np.pi))
    ni = nf.astype(jnp.int32)
    x = ((ang - nf * np.float32(1.5703125))
         - nf * np.float32(4.837512969970703125e-4)) \
        - nf * np.float32(7.54978995489188216e-8)
    z = x * x
    sp = ((np.float32(-1.9515295891e-4) * z + np.float32(8.3321608736e-3)) * z
          + np.float32(-1.6666654611e-1)) * z * x + x
    cp = ((np.float32(2.443315711809948e-5) * z
           + np.float32(-1.388731625493765e-3)) * z
          + np.float32(4.166664568298827e-2)) * z * z \
        - np.float32(0.5) * z + np.float32(1.0)
    q0 = (ni & 1) != 0
    q1 = (ni & 2) != 0
    s = jnp.where(q0, cp, sp)
    s = jnp.where(q1, -s, s)
    c = jnp.where(q0, sp, cp)
    c = jnp.where(((ni + 1) & 2) != 0, -c, c)
    return s, c


def _node_stage(x, W_a1, b_a1, Wg):
    N, C = x.shape
    BN = 1000
    grid = (N // BN,)

    def body(x_ref, wa1_ref, ba1_ref, wg_ref, h_ref, t_ref):
        hb = x_ref[...] @ wa1_ref[...] + ba1_ref[...]
        h_ref[...] = hb
        g = hb @ wg_ref[...]
        t_ref[...] = jnp.concatenate([g, jnp.zeros_like(g)], axis=1)

    return pl.pallas_call(
        body,
        grid=grid,
        in_specs=[
            pl.BlockSpec((BN, C), lambda i: (i, 0)),
            pl.BlockSpec((C, C), lambda i: (0, 0)),
            pl.BlockSpec((1, C), lambda i: (0, 0)),
            pl.BlockSpec((C, 64), lambda i: (0, 0)),
        ],
        out_specs=[
            pl.BlockSpec((BN, C), lambda i: (i, 0)),
            pl.BlockSpec((BN, C), lambda i: (i, 0)),
        ],
        out_shape=[
            jax.ShapeDtypeStruct((N, C), jnp.float32),
            jax.ShapeDtypeStruct((N, C), jnp.float32),
        ],
    )(x, W_a1, b_a1, Wg)


def _sc_gather(src, dst, T, px, py, pz):
    E = src.shape[0]
    N, C = T.shape
    EPW = E // NW
    NCHK = EPW // CH
    mesh = plsc.VectorSubcoreMesh(
        core_axis_name="c", subcore_axis_name="s", num_cores=NC, num_subcores=NS
    )

    @functools.partial(
        pl.kernel,
        mesh=mesh,
        out_type=jax.ShapeDtypeStruct((E, C), jnp.float32),
        compiler_params=pltpu.CompilerParams(needs_layout_passes=False),
        scratch_types=[
            pltpu.VMEM((EPW,), jnp.int32),
            pltpu.VMEM((EPW,), jnp.int32),
            pltpu.VMEM((N,), jnp.float32),
            pltpu.VMEM((N,), jnp.float32),
            pltpu.VMEM((N,), jnp.float32),
            pltpu.VMEM((CH, C), jnp.float32),
            pltpu.VMEM((CH, C), jnp.float32),
            pltpu.SemaphoreType.DMA,
            pltpu.SemaphoreType.DMA,
        ],
    )
    def k(src_hbm, dst_hbm, t_hbm, px_hbm, py_hbm, pz_hbm, tj_out,
          src_v, dst_v, px_v, py_v, pz_v, tjbuf0, tjbuf1, sem0, sem1):
        w = lax.axis_index("c") * NS + lax.axis_index("s")
        base = pl.multiple_of(w * EPW, 8)
        pltpu.sync_copy(src_hbm.at[pl.ds(base, EPW)], src_v)
        pltpu.sync_copy(dst_hbm.at[pl.ds(base, EPW)], dst_v)
        pltpu.sync_copy(px_hbm, px_v)
        pltpu.sync_copy(py_hbm, py_v)
        pltpu.sync_copy(pz_hbm, pz_v)
        lane = lax.iota(jnp.int32, 16)
        bufs = (tjbuf0, tjbuf1)
        sems = (sem0, sem1)

        def start(j, b):
            ch0 = pl.multiple_of(j * CH, 8)
            pltpu.async_copy(t_hbm.at[src_v.at[pl.ds(ch0, CH)]], bufs[b],
                             sems[b])

        def finish(j, b):
            ch0 = pl.multiple_of(j * CH, 8)
            pltpu.make_async_copy(t_hbm.at[src_v.at[pl.ds(ch0, CH)]], bufs[b],
                                  sems[b]).wait()
            buf = bufs[b]
            for gi in range(CH // 16):
                off = pl.multiple_of(j * CH + gi * 16, 8)
                s16 = src_v[pl.ds(off, 16)]
                d16 = dst_v[pl.ds(off, 16)]
                dx = plsc.load_gather(px_v, [d16]) - plsc.load_gather(px_v, [s16])
                dy = plsc.load_gather(py_v, [d16]) - plsc.load_gather(py_v, [s16])
                dz = plsc.load_gather(pz_v, [d16]) - plsc.load_gather(pz_v, [s16])
                r2 = dx * dx + dy * dy + dz * dz
                for cix in range(16):
                    plsc.store_scatter(
                        buf, [gi * 16 + lane, jnp.full((16,), 64 + cix,
                                                       jnp.int32)], r2)
            pltpu.sync_copy(
                buf, tj_out.at[pl.ds(pl.multiple_of(base + j * CH, 8), CH)])

        start(0, 0)
        start(1, 1)

        @pl.loop(0, NCHK - 1, step=2)
        def _(j):
            finish(j, 0)
            start(j + 2, 0)
            finish(j + 1, 1)

            @pl.when(j + 3 < NCHK)
            def _():
                start(j + 3, 1)

        finish(NCHK - 1, 0)

    return k(src, dst, T, px, py, pz)


def _edge_stage(TJ, edge_attr, Wsc, We, bn1, Wn2, bn2, Wn3, bn3, omeg):
    E, C = edge_attr.shape
    MID = Wn2.shape[0]
    BE = 2560
    grid = (E // BE,)

    def body(tj_ref, ea_ref, wsc_ref, we_ref, bn1_ref, w2_ref,
             bn2_ref, w3_ref, bn3_ref, om_ref, out_ref):
        tj = tj_ref[...]
        g = tj[:, :64]
        ang = jnp.sqrt(tj[:, 64:80] * om_ref[...])
        sn, cs = _sincos(ang)
        sc = jnp.concatenate([sn, cs], axis=1)
        pre = sc @ wsc_ref[...] + ea_ref[...] @ we_ref[...] + g + bn1_ref[...]
        h1 = _gelu(pre)
        h2 = _gelu(h1 @ w2_ref[...] + bn2_ref[...])
        out_ref[...] = h2 @ w3_ref[...] + bn3_ref[...]

    return pl.pallas_call(
        body,
        grid=grid,
        in_specs=[
            pl.BlockSpec((BE, C), lambda i: (i, 0)),
            pl.BlockSpec((BE, C), lambda i: (i, 0)),
            pl.BlockSpec((32, MID), lambda i: (0, 0)),
            pl.BlockSpec((C, MID), lambda i: (0, 0)),
            pl.BlockSpec((1, MID), lambda i: (0, 0)),
            pl.BlockSpec((MID, MID), lambda i: (0, 0)),
            pl.BlockSpec((1, MID), lambda i: (0, 0)),
            pl.BlockSpec((MID, C), lambda i: (0, 0)),
            pl.BlockSpec((1, C), lambda i: (0, 0)),
            pl.BlockSpec((1, 16), lambda i: (0, 0)),
        ],
        out_specs=pl.BlockSpec((BE, C), lambda i: (i, 0)),
        out_shape=jax.ShapeDtypeStruct((E, C), jnp.float32),
    )(TJ, edge_attr, Wsc, We, bn1, Wn2, bn2, Wn3, bn3, omeg)


def _sc_scatter(dst3d, msg, zeros):
    E, C = msg.shape
    N = zeros.shape[0]
    EPW = E // NW
    NCHK = EPW // CH
    RPT = 640
    TAIL = N - RPT * (NS - 1)
    mesh = plsc.VectorSubcoreMesh(
        core_axis_name="c", subcore_axis_name="s", num_cores=NC, num_subcores=NS
    )

    @functools.partial(
        pl.kernel,
        mesh=mesh,
        compiler_params=pltpu.CompilerParams(needs_layout_passes=False),
        out_type=jax.ShapeDtypeStruct((NC * N, C), jnp.float32),
        scratch_types=[
            pltpu.VMEM((NCHK, CH), jnp.int32),
            pltpu.VMEM((CH, C), jnp.float32),
            pltpu.VMEM((CH, C), jnp.float32),
            pltpu.VMEM_SHARED((N, C), jnp.float32),
            pltpu.SemaphoreType.DMA,
            pltpu.SemaphoreType.DMA,
        ],
    )
    def k(dst3d_hbm, msg_hbm, z_hbm, agg_out, idx_v, mbuf0, mbuf1, agg_sh,
          sem0, sem1):
        c = lax.axis_index("c")
        s = lax.axis_index("s")
        w = c * NS + s

        @pl.when(s < NS - 1)
        def _():
            r0 = pl.multiple_of(s * RPT, 8)
            pltpu.sync_copy(z_hbm.at[pl.ds(r0, RPT)], agg_sh.at[pl.ds(r0, RPT)])

        @pl.when(s == NS - 1)
        def _():
            r0 = RPT * (NS - 1)
            pltpu.sync_copy(z_hbm.at[pl.ds(r0, TAIL)], agg_sh.at[pl.ds(r0, TAIL)])

        pltpu.sync_copy(dst3d_hbm.at[w], idx_v)
        plsc.subcore_barrier()
        bufs = (mbuf0, mbuf1)
        sems = (sem0, sem1)

        def start(j, b):
            e0 = pl.multiple_of(w * EPW + j * CH, 8)
            pltpu.async_copy(msg_hbm.at[pl.ds(e0, CH)], bufs[b], sems[b])

        def finish(j, b):
            e0 = pl.multiple_of(w * EPW + j * CH, 8)
            pltpu.make_async_copy(msg_hbm.at[pl.ds(e0, CH)], bufs[b],
                                  sems[b]).wait()
            pltpu.sync_copy(bufs[b], agg_sh.at[idx_v.at[j]], add=True)

        start(0, 0)
        start(1, 1)

        @pl.loop(0, NCHK - 1, step=2)
        def _(j):
            finish(j, 0)
            start(j + 2, 0)
            finish(j + 1, 1)

            @pl.when(j + 3 < NCHK)
            def _():
                start(j + 3, 1)

        finish(NCHK - 1, 0)
        plsc.subcore_barrier()

        @pl.when(s < NS - 1)
        def _():
            r0 = pl.multiple_of(s * RPT, 8)
            o0 = pl.multiple_of(c * N + s * RPT, 8)
            pltpu.sync_copy(agg_sh.at[pl.ds(r0, RPT)], agg_out.at[pl.ds(o0, RPT)])

        @pl.when(s == NS - 1)
        def _():
            r0 = RPT * (NS - 1)
            o0 = pl.multiple_of(c * N + r0, 8)
            pltpu.sync_copy(agg_sh.at[pl.ds(r0, TAIL)], agg_out.at[pl.ds(o0, TAIL)])

    return k(dst3d, msg, zeros)


def _final_stage(h, aggp, W_a2, b_a2, W_a3, b_a3):
    N, C = h.shape
    BN = 1000
    grid = (N // BN,)

    def body(h_ref, a0_ref, a1_ref, wa2_ref, ba2_ref, wa3_ref, ba3_ref, out_ref):
        hb = h_ref[...] + a0_ref[...] + a1_ref[...]
        t = jnp.maximum(hb @ wa2_ref[...] + ba2_ref[...], 0.0)
        out_ref[...] = t @ wa3_ref[...] + ba3_ref[...]

    nb = N // BN
    return pl.pallas_call(
        body,
        grid=grid,
        in_specs=[
            pl.BlockSpec((BN, C), lambda i: (i, 0)),
            pl.BlockSpec((BN, C), lambda i: (i, 0)),
            pl.BlockSpec((BN, C), lambda i: (i + nb, 0)),
            pl.BlockSpec((C, C), lambda i: (0, 0)),
            pl.BlockSpec((1, C), lambda i: (0, 0)),
            pl.BlockSpec((C, C), lambda i: (0, 0)),
            pl.BlockSpec((1, C), lambda i: (0, 0)),
        ],
        out_specs=pl.BlockSpec((BN, C), lambda i: (i, 0)),
        out_shape=jax.ShapeDtypeStruct((N, C), jnp.float32),
    )(h, aggp, aggp, W_a2, b_a2, W_a3, b_a3)


def kernel(x, edge_index, edge_attr, x_pos, W_a1, b_a1, Wn1, bn1, Wn2, bn2,
           Wn3, bn3, W_a2, b_a2, W_a3, b_a3):
    N, C = x.shape
    E = edge_index.shape[1]
    NF = 16
    n_channels = 128
    omeg = jnp.asarray(
        [10.0 * (float(n_channels) ** (1.0 - 2.0 * i / NF)) for i in range(NF)],
        jnp.float32).reshape(1, NF)

    src = edge_index[0]
    dst = edge_index[1]
    Wsc = Wn1[: 2 * NF]
    We = Wn1[2 * NF: 2 * NF + C]
    Wg = Wn1[2 * NF + C:]

    h, T = _node_stage(x, W_a1, b_a1.reshape(1, C), Wg)
    TJ = _sc_gather(src, dst, T, x_pos[:, 0], x_pos[:, 1], x_pos[:, 2])
    msg = _edge_stage(TJ, edge_attr, Wsc, We, bn1.reshape(1, -1), Wn2,
                      bn2.reshape(1, -1), Wn3, bn3.reshape(1, C), omeg * omeg)
    aggp = _sc_scatter(dst.reshape(NW, E // (NW * CH), CH), msg,
                       jnp.zeros((N, C), jnp.float32))
    return _final_stage(h, aggp, W_a2, b_a2.reshape(1, C), W_a3,
                        b_a3.reshape(1, C))


# speedup vs baseline: 5.9202x; 1.1857x over previous
import functools

import jax
import jax.numpy as jnp
import numpy as np
from jax import lax
from jax.experimental import pallas as pl
from jax.experimental.pallas import tpu as pltpu
from jax.experimental.pallas import tpu_sc as plsc

NC = 2
NS = 16
NW = NC * NS
CH = 80


def _gelu(u):
    return 0.5 * u * (1.0 + lax.erf(u * np.float32(1.0 / np.sqrt(2.0))))


def _sincos(ang):
    nf = jnp.round(ang * np.float32(2.0 / np.pi))
    ni = nf.astype(jnp.int32)
    x = ((ang - nf * np.float32(1.5703125))
         - nf * np.float32(4.837512969970703125e-4)) \
        - nf * np.float32(7.54978995489188216e-8)
    z = x * x
    sp = ((np.float32(-1.9515295891e-4) * z + np.float32(8.3321608736e-3)) * z
          + np.float32(-1.6666654611e-1)) * z * x + x
    cp = ((np.float32(2.443315711809948e-5) * z
           + np.float32(-1.388731625493765e-3)) * z
          + np.float32(4.166664568298827e-2)) * z * z \
        - np.float32(0.5) * z + np.float32(1.0)
    q0 = (ni & 1) != 0
    q1 = (ni & 2) != 0
    s = jnp.where(q0, cp, sp)
    s = jnp.where(q1, -s, s)
    c = jnp.where(q0, sp, cp)
    c = jnp.where(((ni + 1) & 2) != 0, -c, c)
    return s, c


def _node_stage(x, W_a1, b_a1, Wg):
    N, C = x.shape
    BN = 1000
    grid = (N // BN,)

    def body(x_ref, wa1_ref, ba1_ref, wg_ref, h_ref, t_ref):
        hb = x_ref[...] @ wa1_ref[...] + ba1_ref[...]
        h_ref[...] = hb
        g = hb @ wg_ref[...]
        t_ref[...] = jnp.concatenate([g, jnp.zeros_like(g)], axis=1)

    return pl.pallas_call(
        body,
        grid=grid,
        in_specs=[
            pl.BlockSpec((BN, C), lambda i: (i, 0)),
            pl.BlockSpec((C, C), lambda i: (0, 0)),
            pl.BlockSpec((1, C), lambda i: (0, 0)),
            pl.BlockSpec((C, 64), lambda i: (0, 0)),
        ],
        out_specs=[
            pl.BlockSpec((BN, C), lambda i: (i, 0)),
            pl.BlockSpec((BN, C), lambda i: (i, 0)),
        ],
        out_shape=[
            jax.ShapeDtypeStruct((N, C), jnp.float32),
            jax.ShapeDtypeStruct((N, C), jnp.float32),
        ],
    )(x, W_a1, b_a1, Wg)


def _sc_gather(src, dst, T, px, py, pz):
    E = src.shape[0]
    N, C = T.shape
    EPW = E // NW
    NCHK = EPW // CH
    mesh = plsc.VectorSubcoreMesh(
        core_axis_name="c", subcore_axis_name="s", num_cores=NC, num_subcores=NS
    )

    @functools.partial(
        pl.kernel,
        mesh=mesh,
        out_type=jax.ShapeDtypeStruct((E, C), jnp.float32),
        compiler_params=pltpu.CompilerParams(needs_layout_passes=False),
        scratch_types=[
            pltpu.VMEM((EPW,), jnp.int32),
            pltpu.VMEM((EPW,), jnp.int32),
            pltpu.VMEM((N,), jnp.float32),
            pltpu.VMEM((N,), jnp.float32),
            pltpu.VMEM((N,), jnp.float32),
            pltpu.VMEM((CH, C), jnp.float32),
            pltpu.VMEM((CH, C), jnp.float32),
            pltpu.SemaphoreType.DMA,
            pltpu.SemaphoreType.DMA,
        ],
    )
    def k(src_hbm, dst_hbm, t_hbm, px_hbm, py_hbm, pz_hbm, tj_out,
          src_v, dst_v, px_v, py_v, pz_v, tjbuf0, tjbuf1, sem0, sem1):
        w = lax.axis_index("c") * NS + lax.axis_index("s")
        base = pl.multiple_of(w * EPW, 8)
        pltpu.sync_copy(src_hbm.at[pl.ds(base, EPW)], src_v)
        pltpu.sync_copy(dst_hbm.at[pl.ds(base, EPW)], dst_v)
        pltpu.sync_copy(px_hbm, px_v)
        pltpu.sync_copy(py_hbm, py_v)
        pltpu.sync_copy(pz_hbm, pz_v)
        lane = lax.iota(jnp.int32, 16)
        bufs = (tjbuf0, tjbuf1)
        sems = (sem0, sem1)

        def start(j, b):
            ch0 = pl.multiple_of(j * CH, 8)
            pltpu.async_copy(t_hbm.at[src_v.at[pl.ds(ch0, CH)]], bufs[b],
                             sems[b])

        def finish(j, b):
            ch0 = pl.multiple_of(j * CH, 8)
            pltpu.make_async_copy(t_hbm.at[src_v.at[pl.ds(ch0, CH)]], bufs[b],
                                  sems[b]).wait()
            buf = bufs[b]
            for gi in range(CH // 16):
                off = pl.multiple_of(j * CH + gi * 16, 8)
                s16 = src_v[pl.ds(off, 16)]
                d16 = dst_v[pl.ds(off, 16)]
                dx = plsc.load_gather(px_v, [d16]) - plsc.load_gather(px_v, [s16])
                dy = plsc.load_gather(py_v, [d16]) - plsc.load_gather(py_v, [s16])
                dz = plsc.load_gather(pz_v, [d16]) - plsc.load_gather(pz_v, [s16])
                r2 = dx * dx + dy * dy + dz * dz
                for cix in range(16):
                    plsc.store_scatter(
                        buf, [gi * 16 + lane, jnp.full((16,), 64 + cix,
                                                       jnp.int32)], r2)
            pltpu.sync_copy(
                buf, tj_out.at[pl.ds(pl.multiple_of(base + j * CH, 8), CH)])

        start(0, 0)
        start(1, 1)

        @pl.loop(0, NCHK - 1, step=2)
        def _(j):
            finish(j, 0)

            @pl.when(j + 2 < NCHK)
            def _():
                start(j + 2, 0)

            finish(j + 1, 1)

            @pl.when(j + 3 < NCHK)
            def _():
                start(j + 3, 1)

        if NCHK % 2 == 1:
            finish(NCHK - 1, 0)

    return k(src, dst, T, px, py, pz)


def _edge_stage(TJ, edge_attr, Wsc, We, bn1, Wn2, bn2, Wn3, bn3, omeg,
                blk_off=0):
    E = TJ.shape[0]
    C = edge_attr.shape[1]
    MID = Wn2.shape[0]
    BE = 2560
    grid = (E // BE,)

    def body(tj_ref, ea_ref, wsc_ref, we_ref, bn1_ref, w2_ref,
             bn2_ref, w3_ref, bn3_ref, om_ref, out_ref):
        tj = tj_ref[...]
        g = tj[:, :64]
        ang = jnp.sqrt(tj[:, 64:80] * om_ref[...])
        sn, cs = _sincos(ang)
        sc = jnp.concatenate([sn, cs], axis=1)
        pre = sc @ wsc_ref[...] + ea_ref[...] @ we_ref[...] + g + bn1_ref[...]
        h1 = _gelu(pre)
        h2 = _gelu(h1 @ w2_ref[...] + bn2_ref[...])
        out_ref[...] = h2 @ w3_ref[...] + bn3_ref[...]

    return pl.pallas_call(
        body,
        grid=grid,
        in_specs=[
            pl.BlockSpec((BE, C), lambda i: (i, 0)),
            pl.BlockSpec((BE, C), lambda i: (i + blk_off, 0)),
            pl.BlockSpec((32, MID), lambda i: (0, 0)),
            pl.BlockSpec((C, MID), lambda i: (0, 0)),
            pl.BlockSpec((1, MID), lambda i: (0, 0)),
            pl.BlockSpec((MID, MID), lambda i: (0, 0)),
            pl.BlockSpec((1, MID), lambda i: (0, 0)),
            pl.BlockSpec((MID, C), lambda i: (0, 0)),
            pl.BlockSpec((1, C), lambda i: (0, 0)),
            pl.BlockSpec((1, 16), lambda i: (0, 0)),
        ],
        out_specs=pl.BlockSpec((BE, C), lambda i: (i, 0)),
        out_shape=jax.ShapeDtypeStruct((E, C), jnp.float32),
    )(TJ, edge_attr, Wsc, We, bn1, Wn2, bn2, Wn3, bn3, omeg)


def _sc_scatter(dst3d, msg, zeros):
    E, C = msg.shape
    N = zeros.shape[0]
    EPW = E // NW
    NCHK = EPW // CH
    RPT = 640
    TAIL = N - RPT * (NS - 1)
    mesh = plsc.VectorSubcoreMesh(
        core_axis_name="c", subcore_axis_name="s", num_cores=NC, num_subcores=NS
    )

    @functools.partial(
        pl.kernel,
        mesh=mesh,
        compiler_params=pltpu.CompilerParams(needs_layout_passes=False),
        out_type=jax.ShapeDtypeStruct((NC * N, C), jnp.float32),
        scratch_types=[
            pltpu.VMEM((NCHK, CH), jnp.int32),
            pltpu.VMEM((CH, C), jnp.float32),
            pltpu.VMEM((CH, C), jnp.float32),
            pltpu.VMEM_SHARED((N, C), jnp.float32),
            pltpu.SemaphoreType.DMA,
            pltpu.SemaphoreType.DMA,
        ],
    )
    def k(dst3d_hbm, msg_hbm, z_hbm, agg_out, idx_v, mbuf0, mbuf1, agg_sh,
          sem0, sem1):
        c = lax.axis_index("c")
        s = lax.axis_index("s")
        w = c * NS + s

        @pl.when(s < NS - 1)
        def _():
            r0 = pl.multiple_of(s * RPT, 8)
            pltpu.sync_copy(z_hbm.at[pl.ds(r0, RPT)], agg_sh.at[pl.ds(r0, RPT)])

        @pl.when(s == NS - 1)
        def _():
            r0 = RPT * (NS - 1)
            pltpu.sync_copy(z_hbm.at[pl.ds(r0, TAIL)], agg_sh.at[pl.ds(r0, TAIL)])

        pltpu.sync_copy(dst3d_hbm.at[w], idx_v)
        plsc.subcore_barrier()
        bufs = (mbuf0, mbuf1)
        sems = (sem0, sem1)

        def start(j, b):
            e0 = pl.multiple_of(w * EPW + j * CH, 8)
            pltpu.async_copy(msg_hbm.at[pl.ds(e0, CH)], bufs[b], sems[b])

        def finish(j, b):
            e0 = pl.multiple_of(w * EPW + j * CH, 8)
            pltpu.make_async_copy(msg_hbm.at[pl.ds(e0, CH)], bufs[b],
                                  sems[b]).wait()
            pltpu.sync_copy(bufs[b], agg_sh.at[idx_v.at[j]], add=True)

        start(0, 0)
        start(1, 1)

        @pl.loop(0, NCHK - 1, step=2)
        def _(j):
            finish(j, 0)

            @pl.when(j + 2 < NCHK)
            def _():
                start(j + 2, 0)

            finish(j + 1, 1)

            @pl.when(j + 3 < NCHK)
            def _():
                start(j + 3, 1)

        if NCHK % 2 == 1:
            finish(NCHK - 1, 0)
        plsc.subcore_barrier()

        @pl.when(s < NS - 1)
        def _():
            r0 = pl.multiple_of(s * RPT, 8)
            o0 = pl.multiple_of(c * N + s * RPT, 8)
            pltpu.sync_copy(agg_sh.at[pl.ds(r0, RPT)], agg_out.at[pl.ds(o0, RPT)])

        @pl.when(s == NS - 1)
        def _():
            r0 = RPT * (NS - 1)
            o0 = pl.multiple_of(c * N + r0, 8)
            pltpu.sync_copy(agg_sh.at[pl.ds(r0, TAIL)], agg_out.at[pl.ds(o0, TAIL)])

    return k(dst3d, msg, zeros)


def _final_stage(h, aggpA, aggpB, W_a2, b_a2, W_a3, b_a3):
    N, C = h.shape
    BN = 1000
    grid = (N // BN,)

    def body(h_ref, a0_ref, a1_ref, a2_ref, a3_ref, wa2_ref, ba2_ref,
             wa3_ref, ba3_ref, out_ref):
        hb = (h_ref[...] + a0_ref[...] + a1_ref[...]
              + a2_ref[...] + a3_ref[...])
        t = jnp.maximum(hb @ wa2_ref[...] + ba2_ref[...], 0.0)
        out_ref[...] = t @ wa3_ref[...] + ba3_ref[...]

    nb = N // BN
    return pl.pallas_call(
        body,
        grid=grid,
        in_specs=[
            pl.BlockSpec((BN, C), lambda i: (i, 0)),
            pl.BlockSpec((BN, C), lambda i: (i, 0)),
            pl.BlockSpec((BN, C), lambda i: (i + nb, 0)),
            pl.BlockSpec((BN, C), lambda i: (i, 0)),
            pl.BlockSpec((BN, C), lambda i: (i + nb, 0)),
            pl.BlockSpec((C, C), lambda i: (0, 0)),
            pl.BlockSpec((1, C), lambda i: (0, 0)),
            pl.BlockSpec((C, C), lambda i: (0, 0)),
            pl.BlockSpec((1, C), lambda i: (0, 0)),
        ],
        out_specs=pl.BlockSpec((BN, C), lambda i: (i, 0)),
        out_shape=jax.ShapeDtypeStruct((N, C), jnp.float32),
    )(h, aggpA, aggpA, aggpB, aggpB, W_a2, b_a2, W_a3, b_a3)


def kernel(x, edge_index, edge_attr, x_pos, W_a1, b_a1, Wn1, bn1, Wn2, bn2,
           Wn3, bn3, W_a2, b_a2, W_a3, b_a3):
    N, C = x.shape
    E = edge_index.shape[1]
    NF = 16
    n_channels = 128
    omeg = jnp.asarray(
        [10.0 * (float(n_channels) ** (1.0 - 2.0 * i / NF)) for i in range(NF)],
        jnp.float32).reshape(1, NF)

    src = edge_index[0]
    dst = edge_index[1]
    Wsc = Wn1[: 2 * NF]
    We = Wn1[2 * NF: 2 * NF + C]
    Wg = Wn1[2 * NF + C:]

    h, T = _node_stage(x, W_a1, b_a1.reshape(1, C), Wg)
    px, py, pz = x_pos[:, 0], x_pos[:, 1], x_pos[:, 2]
    om2 = omeg * omeg
    zer = jnp.zeros((N, C), jnp.float32)
    EA = (E // 2 // (NW * CH)) * (NW * CH)
    srcA, dstA = src[:EA], dst[:EA]
    srcB, dstB = src[EA:], dst[EA:]
    TJA = _sc_gather(srcA, dstA, T, px, py, pz)
    TJB = _sc_gather(srcB, dstB, T, px, py, pz)
    bn1r, bn2r, bn3r = bn1.reshape(1, -1), bn2.reshape(1, -1), bn3.reshape(1, C)
    msgA = _edge_stage(TJA, edge_attr, Wsc, We, bn1r, Wn2, bn2r, Wn3, bn3r,
                       om2, blk_off=0)
    msgB = _edge_stage(TJB, edge_attr, Wsc, We, bn1r, Wn2, bn2r, Wn3, bn3r,
                       om2, blk_off=EA // 2560)
    aggA = _sc_scatter(dstA.reshape(NW, EA // (NW * CH), CH), msgA, zer)
    aggB = _sc_scatter(dstB.reshape(NW, (E - EA) // (NW * CH), CH), msgB, zer)
    return _final_stage(h, aggA, aggB, W_a2, b_a2.reshape(1, C), W_a3,
                        b_a3.reshape(1, C))


# speedup vs baseline: 6.0973x; 1.0299x over previous
import functools

import jax
import jax.numpy as jnp
import numpy as np
from jax import lax
from jax.experimental import pallas as pl
from jax.experimental.pallas import tpu as pltpu
from jax.experimental.pallas import tpu_sc as plsc

NC = 2
NS = 16
NW = NC * NS
CH = 80


def _sincos(ang):
    nf = jnp.round(ang * np.float32(2.0 / np.pi))
    ni = nf.astype(jnp.int32)
    x = ((ang - nf * np.float32(1.5703125))
         - nf * np.float32(4.837512969970703125e-4)) \
        - nf * np.float32(7.54978995489188216e-8)
    z = x * x
    sp = ((np.float32(-1.9515295891e-4) * z + np.float32(8.3321608736e-3)) * z
          + np.float32(-1.6666654611e-1)) * z * x + x
    cp = ((np.float32(2.443315711809948e-5) * z
           + np.float32(-1.388731625493765e-3)) * z
          + np.float32(4.166664568298827e-2)) * z * z \
        - np.float32(0.5) * z + np.float32(1.0)
    q0 = (ni & 1) != 0
    ssel = jnp.where(q0, cp, sp)
    csel = jnp.where(q0, sp, cp)
    sflip = (ni & 2) << 30
    cflip = ((ni + 1) & 2) << 30
    s = lax.bitcast_convert_type(
        lax.bitcast_convert_type(ssel, jnp.int32) ^ sflip, jnp.float32)
    c = lax.bitcast_convert_type(
        lax.bitcast_convert_type(csel, jnp.int32) ^ cflip, jnp.float32)
    return s, c


def _node_stage(x, W_a1, b_a1, Wg):
    N, C = x.shape
    BN = 1000
    grid = (N // BN,)

    def body(x_ref, wa1_ref, ba1_ref, wg_ref, h_ref, t_ref):
        hb = x_ref[...] @ wa1_ref[...] + ba1_ref[...]
        h_ref[...] = hb
        g = hb @ wg_ref[...]
        t_ref[...] = jnp.concatenate([g, jnp.zeros_like(g)], axis=1)

    return pl.pallas_call(
        body,
        grid=grid,
        in_specs=[
            pl.BlockSpec((BN, C), lambda i: (i, 0)),
            pl.BlockSpec((C, C), lambda i: (0, 0)),
            pl.BlockSpec((1, C), lambda i: (0, 0)),
            pl.BlockSpec((C, 64), lambda i: (0, 0)),
        ],
        out_specs=[
            pl.BlockSpec((BN, C), lambda i: (i, 0)),
            pl.BlockSpec((BN, C), lambda i: (i, 0)),
        ],
        out_shape=[
            jax.ShapeDtypeStruct((N, C), jnp.float32),
            jax.ShapeDtypeStruct((N, C), jnp.float32),
        ],
    )(x, W_a1, b_a1, Wg)


def _sc_gather(src, dst, T, px, py, pz):
    E = src.shape[0]
    N, C = T.shape
    EPW = E // NW
    NCHK = EPW // CH
    mesh = plsc.VectorSubcoreMesh(
        core_axis_name="c", subcore_axis_name="s", num_cores=NC, num_subcores=NS
    )

    @functools.partial(
        pl.kernel,
        mesh=mesh,
        out_type=jax.ShapeDtypeStruct((E, C), jnp.float32),
        compiler_params=pltpu.CompilerParams(needs_layout_passes=False),
        scratch_types=[
            pltpu.VMEM((EPW,), jnp.int32),
            pltpu.VMEM((EPW,), jnp.int32),
            pltpu.VMEM((N,), jnp.float32),
            pltpu.VMEM((N,), jnp.float32),
            pltpu.VMEM((N,), jnp.float32),
            pltpu.VMEM((CH, C), jnp.float32),
            pltpu.VMEM((CH, C), jnp.float32),
            pltpu.SemaphoreType.DMA,
            pltpu.SemaphoreType.DMA,
        ],
    )
    def k(src_hbm, dst_hbm, t_hbm, px_hbm, py_hbm, pz_hbm, tj_out,
          src_v, dst_v, px_v, py_v, pz_v, tjbuf0, tjbuf1, sem0, sem1):
        w = lax.axis_index("c") * NS + lax.axis_index("s")
        base = pl.multiple_of(w * EPW, 8)
        pltpu.sync_copy(src_hbm.at[pl.ds(base, EPW)], src_v)
        pltpu.sync_copy(dst_hbm.at[pl.ds(base, EPW)], dst_v)
        pltpu.sync_copy(px_hbm, px_v)
        pltpu.sync_copy(py_hbm, py_v)
        pltpu.sync_copy(pz_hbm, pz_v)
        lane = lax.iota(jnp.int32, 16)
        bufs = (tjbuf0, tjbuf1)
        sems = (sem0, sem1)

        def start(j, b):
            ch0 = pl.multiple_of(j * CH, 8)
            pltpu.async_copy(t_hbm.at[src_v.at[pl.ds(ch0, CH)]], bufs[b],
                             sems[b])

        def finish(j, b):
            ch0 = pl.multiple_of(j * CH, 8)
            pltpu.make_async_copy(t_hbm.at[src_v.at[pl.ds(ch0, CH)]], bufs[b],
                                  sems[b]).wait()
            buf = bufs[b]
            for gi in range(CH // 16):
                off = pl.multiple_of(j * CH + gi * 16, 8)
                s16 = src_v[pl.ds(off, 16)]
                d16 = dst_v[pl.ds(off, 16)]
                dx = plsc.load_gather(px_v, [d16]) - plsc.load_gather(px_v, [s16])
                dy = plsc.load_gather(py_v, [d16]) - plsc.load_gather(py_v, [s16])
                dz = plsc.load_gather(pz_v, [d16]) - plsc.load_gather(pz_v, [s16])
                r2 = dx * dx + dy * dy + dz * dz
                for cix in range(16):
                    plsc.store_scatter(
                        buf, [gi * 16 + lane, jnp.full((16,), 64 + cix,
                                                       jnp.int32)], r2)
            pltpu.sync_copy(
                buf, tj_out.at[pl.ds(pl.multiple_of(base + j * CH, 8), CH)])

        start(0, 0)
        start(1, 1)

        @pl.loop(0, NCHK - 1, step=2)
        def _(j):
            finish(j, 0)

            @pl.when(j + 2 < NCHK)
            def _():
                start(j + 2, 0)

            finish(j + 1, 1)

            @pl.when(j + 3 < NCHK)
            def _():
                start(j + 3, 1)

        if NCHK % 2 == 1:
            finish(NCHK - 1, 0)

    return k(src, dst, T, px, py, pz)


def _edge_stage(TJ, edge_attr, Wsc, We, bn1, Wn2, bn2, Wn3, bn3, omeg,
                blk_off=0):
    E = TJ.shape[0]
    C = edge_attr.shape[1]
    MID = Wn2.shape[0]
    BE = 2560
    grid = (E // BE,)

    def body(tj_ref, ea_ref, wsc_ref, we_ref, bn1_ref, w2_ref,
             bn2_ref, w3_ref, bn3_ref, om_ref, out_ref):
        tj = tj_ref[...]
        g = tj[:, :64]
        ang = jnp.sqrt(tj[:, 64:80] * om_ref[...])
        sn, cs = _sincos(ang)
        sc = jnp.concatenate([sn, cs], axis=1)
        p1 = (sc @ wsc_ref[...]
              + ea_ref[...] @ we_ref[...] + g + bn1_ref[...])
        q1 = p1 * lax.erf(p1) + p1
        p2 = q1 @ w2_ref[...] + bn2_ref[...]
        q2 = p2 * lax.erf(p2) + p2
        out_ref[...] = q2 @ w3_ref[...] + bn3_ref[...]

    return pl.pallas_call(
        body,
        grid=grid,
        in_specs=[
            pl.BlockSpec((BE, C), lambda i: (i, 0)),
            pl.BlockSpec((BE, C), lambda i: (i + blk_off, 0)),
            pl.BlockSpec((32, MID), lambda i: (0, 0)),
            pl.BlockSpec((C, MID), lambda i: (0, 0)),
            pl.BlockSpec((1, MID), lambda i: (0, 0)),
            pl.BlockSpec((MID, MID), lambda i: (0, 0)),
            pl.BlockSpec((1, MID), lambda i: (0, 0)),
            pl.BlockSpec((MID, C), lambda i: (0, 0)),
            pl.BlockSpec((1, C), lambda i: (0, 0)),
            pl.BlockSpec((1, 16), lambda i: (0, 0)),
        ],
        out_specs=pl.BlockSpec((BE, C), lambda i: (i, 0)),
        out_shape=jax.ShapeDtypeStruct((E, C), jnp.float32),
    )(TJ, edge_attr, Wsc, We, bn1, Wn2, bn2, Wn3, bn3, omeg)


def _sc_scatter(dst3d, msg, zeros):
    E, C = msg.shape
    N = zeros.shape[0]
    EPW = E // NW
    NCHK = EPW // CH
    RPT = 640
    TAIL = N - RPT * (NS - 1)
    mesh = plsc.VectorSubcoreMesh(
        core_axis_name="c", subcore_axis_name="s", num_cores=NC, num_subcores=NS
    )

    @functools.partial(
        pl.kernel,
        mesh=mesh,
        compiler_params=pltpu.CompilerParams(needs_layout_passes=False),
        out_type=jax.ShapeDtypeStruct((NC * N, C), jnp.float32),
        scratch_types=[
            pltpu.VMEM((NCHK, CH), jnp.int32),
            pltpu.VMEM((CH, C), jnp.float32),
            pltpu.VMEM((CH, C), jnp.float32),
            pltpu.VMEM_SHARED((N, C), jnp.float32),
            pltpu.SemaphoreType.DMA,
            pltpu.SemaphoreType.DMA,
        ],
    )
    def k(dst3d_hbm, msg_hbm, z_hbm, agg_out, idx_v, mbuf0, mbuf1, agg_sh,
          sem0, sem1):
        c = lax.axis_index("c")
        s = lax.axis_index("s")
        w = c * NS + s

        @pl.when(s < NS - 1)
        def _():
            r0 = pl.multiple_of(s * RPT, 8)
            pltpu.sync_copy(z_hbm.at[pl.ds(r0, RPT)], agg_sh.at[pl.ds(r0, RPT)])

        @pl.when(s == NS - 1)
        def _():
            r0 = RPT * (NS - 1)
            pltpu.sync_copy(z_hbm.at[pl.ds(r0, TAIL)], agg_sh.at[pl.ds(r0, TAIL)])

        pltpu.sync_copy(dst3d_hbm.at[w], idx_v)
        plsc.subcore_barrier()
        bufs = (mbuf0, mbuf1)
        sems = (sem0, sem1)

        def start(j, b):
            e0 = pl.multiple_of(w * EPW + j * CH, 8)
            pltpu.async_copy(msg_hbm.at[pl.ds(e0, CH)], bufs[b], sems[b])

        def finish(j, b):
            e0 = pl.multiple_of(w * EPW + j * CH, 8)
            pltpu.make_async_copy(msg_hbm.at[pl.ds(e0, CH)], bufs[b],
                                  sems[b]).wait()
            pltpu.sync_copy(bufs[b], agg_sh.at[idx_v.at[j]], add=True)

        start(0, 0)
        start(1, 1)

        @pl.loop(0, NCHK - 1, step=2)
        def _(j):
            finish(j, 0)

            @pl.when(j + 2 < NCHK)
            def _():
                start(j + 2, 0)

            finish(j + 1, 1)

            @pl.when(j + 3 < NCHK)
            def _():
                start(j + 3, 1)

        if NCHK % 2 == 1:
            finish(NCHK - 1, 0)
        plsc.subcore_barrier()

        @pl.when(s < NS - 1)
        def _():
            r0 = pl.multiple_of(s * RPT, 8)
            o0 = pl.multiple_of(c * N + s * RPT, 8)
            pltpu.sync_copy(agg_sh.at[pl.ds(r0, RPT)], agg_out.at[pl.ds(o0, RPT)])

        @pl.when(s == NS - 1)
        def _():
            r0 = RPT * (NS - 1)
            o0 = pl.multiple_of(c * N + r0, 8)
            pltpu.sync_copy(agg_sh.at[pl.ds(r0, TAIL)], agg_out.at[pl.ds(o0, TAIL)])

    return k(dst3d, msg, zeros)


def _final_stage(h, aggpA, aggpB, W_a2, b_a2, W_a3, b_a3):
    N, C = h.shape
    BN = 1000
    grid = (N // BN,)

    def body(h_ref, a0_ref, a1_ref, a2_ref, a3_ref, wa2_ref, ba2_ref,
             wa3_ref, ba3_ref, out_ref):
        hb = (h_ref[...] + a0_ref[...] + a1_ref[...]
              + a2_ref[...] + a3_ref[...])
        t = jnp.maximum(hb @ wa2_ref[...] + ba2_ref[...], 0.0)
        out_ref[...] = t @ wa3_ref[...] + ba3_ref[...]

    nb = N // BN
    return pl.pallas_call(
        body,
        grid=grid,
        in_specs=[
            pl.BlockSpec((BN, C), lambda i: (i, 0)),
            pl.BlockSpec((BN, C), lambda i: (i, 0)),
            pl.BlockSpec((BN, C), lambda i: (i + nb, 0)),
            pl.BlockSpec((BN, C), lambda i: (i, 0)),
            pl.BlockSpec((BN, C), lambda i: (i + nb, 0)),
            pl.BlockSpec((C, C), lambda i: (0, 0)),
            pl.BlockSpec((1, C), lambda i: (0, 0)),
            pl.BlockSpec((C, C), lambda i: (0, 0)),
            pl.BlockSpec((1, C), lambda i: (0, 0)),
        ],
        out_specs=pl.BlockSpec((BN, C), lambda i: (i, 0)),
        out_shape=jax.ShapeDtypeStruct((N, C), jnp.float32),
    )(h, aggpA, aggpA, aggpB, aggpB, W_a2, b_a2, W_a3, b_a3)


def kernel(x, edge_index, edge_attr, x_pos, W_a1, b_a1, Wn1, bn1, Wn2, bn2,
           Wn3, bn3, W_a2, b_a2, W_a3, b_a3):
    N, C = x.shape
    E = edge_index.shape[1]
    NF = 16
    n_channels = 128
    omeg = jnp.asarray(
        [10.0 * (float(n_channels) ** (1.0 - 2.0 * i / NF)) for i in range(NF)],
        jnp.float32).reshape(1, NF)

    src = edge_index[0]
    dst = edge_index[1]
    rs2 = np.float32(0.7071067811865476)
    Wsc = Wn1[: 2 * NF] * rs2
    We = Wn1[2 * NF: 2 * NF + C] * rs2
    Wg = Wn1[2 * NF + C:] * rs2
    bn1 = bn1 * rs2
    Wn2 = Wn2 * np.float32(0.5)
    bn2 = bn2 * rs2
    Wn3 = Wn3 * rs2

    h, T = _node_stage(x, W_a1, b_a1.reshape(1, C), Wg)
    px, py, pz = x_pos[:, 0], x_pos[:, 1], x_pos[:, 2]
    om2 = omeg * omeg
    zer = jnp.zeros((N, C), jnp.float32)
    EA = (E // 2 // (NW * CH)) * (NW * CH)
    srcA, dstA = src[:EA], dst[:EA]
    srcB, dstB = src[EA:], dst[EA:]
    TJA = _sc_gather(srcA, dstA, T, px, py, pz)
    TJB = _sc_gather(srcB, dstB, T, px, py, pz)
    bn1r, bn2r, bn3r = bn1.reshape(1, -1), bn2.reshape(1, -1), bn3.reshape(1, C)
    msgA = _edge_stage(TJA, edge_attr, Wsc, We, bn1r, Wn2, bn2r, Wn3, bn3r,
                       om2, blk_off=0)
    msgB = _edge_stage(TJB, edge_attr, Wsc, We, bn1r, Wn2, bn2r, Wn3, bn3r,
                       om2, blk_off=EA // 2560)
    aggA = _sc_scatter(dstA.reshape(NW, EA // (NW * CH), CH), msgA, zer)
    aggB = _sc_scatter(dstB.reshape(NW, (E - EA) // (NW * CH), CH), msgB, zer)
    return _final_stage(h, aggA, aggB, W_a2, b_a2.reshape(1, C), W_a3,
                        b_a3.reshape(1, C))


# speedup vs baseline: 6.1127x; 1.0025x over previous
import functools

import jax
import jax.numpy as jnp
import numpy as np
from jax import lax
from jax.experimental import pallas as pl
from jax.experimental.pallas import tpu as pltpu
from jax.experimental.pallas import tpu_sc as plsc

NC = 2
NS = 16
NW = NC * NS
CH = 80


def _sincos(ang):
    nf = jnp.round(ang * np.float32(2.0 / np.pi))
    ni = nf.astype(jnp.int32)
    x = ((ang - nf * np.float32(1.5703125))
         - nf * np.float32(4.837512969970703125e-4)) \
        - nf * np.float32(7.54978995489188216e-8)
    z = x * x
    sp = ((np.float32(-1.9515295891e-4) * z + np.float32(8.3321608736e-3)) * z
          + np.float32(-1.6666654611e-1)) * z * x + x
    cp = ((np.float32(2.443315711809948e-5) * z
           + np.float32(-1.388731625493765e-3)) * z
          + np.float32(4.166664568298827e-2)) * z * z \
        - np.float32(0.5) * z + np.float32(1.0)
    q0 = (ni & 1) != 0
    ssel = jnp.where(q0, cp, sp)
    csel = jnp.where(q0, sp, cp)
    sflip = (ni & 2) << 30
    cflip = ((ni + 1) & 2) << 30
    s = lax.bitcast_convert_type(
        lax.bitcast_convert_type(ssel, jnp.int32) ^ sflip, jnp.float32)
    c = lax.bitcast_convert_type(
        lax.bitcast_convert_type(csel, jnp.int32) ^ cflip, jnp.float32)
    return s, c


def _node_stage(x, W_a1, b_a1, Wg):
    N, C = x.shape
    BN = 1000
    grid = (N // BN,)

    def body(x_ref, wa1_ref, ba1_ref, wg_ref, h_ref, t_ref):
        hb = x_ref[...] @ wa1_ref[...] + ba1_ref[...]
        h_ref[...] = hb
        g = hb @ wg_ref[...]
        t_ref[...] = jnp.concatenate([g, jnp.zeros_like(g)], axis=1)

    return pl.pallas_call(
        body,
        grid=grid,
        in_specs=[
            pl.BlockSpec((BN, C), lambda i: (i, 0)),
            pl.BlockSpec((C, C), lambda i: (0, 0)),
            pl.BlockSpec((1, C), lambda i: (0, 0)),
            pl.BlockSpec((C, 64), lambda i: (0, 0)),
        ],
        out_specs=[
            pl.BlockSpec((BN, C), lambda i: (i, 0)),
            pl.BlockSpec((BN, C), lambda i: (i, 0)),
        ],
        out_shape=[
            jax.ShapeDtypeStruct((N, C), jnp.float32),
            jax.ShapeDtypeStruct((N, C), jnp.float32),
        ],
    )(x, W_a1, b_a1, Wg)


def _sc_gather(src, dst, T, px, py, pz):
    E = src.shape[0]
    N, C = T.shape
    EPW = E // NW
    NCHK = EPW // CH
    mesh = plsc.VectorSubcoreMesh(
        core_axis_name="c", subcore_axis_name="s", num_cores=NC, num_subcores=NS
    )

    @functools.partial(
        pl.kernel,
        mesh=mesh,
        out_type=jax.ShapeDtypeStruct((E, C), jnp.float32),
        compiler_params=pltpu.CompilerParams(needs_layout_passes=False),
        scratch_types=[
            pltpu.VMEM((EPW,), jnp.int32),
            pltpu.VMEM((EPW,), jnp.int32),
            pltpu.VMEM((N,), jnp.float32),
            pltpu.VMEM((N,), jnp.float32),
            pltpu.VMEM((N,), jnp.float32),
            pltpu.VMEM((CH, C), jnp.float32),
            pltpu.VMEM((CH, C), jnp.float32),
            pltpu.SemaphoreType.DMA,
            pltpu.SemaphoreType.DMA,
        ],
    )
    def k(src_hbm, dst_hbm, t_hbm, px_hbm, py_hbm, pz_hbm, tj_out,
          src_v, dst_v, px_v, py_v, pz_v, tjbuf0, tjbuf1, sem0, sem1):
        w = lax.axis_index("c") * NS + lax.axis_index("s")
        base = pl.multiple_of(w * EPW, 8)
        pltpu.sync_copy(src_hbm.at[pl.ds(base, EPW)], src_v)
        pltpu.sync_copy(dst_hbm.at[pl.ds(base, EPW)], dst_v)
        pltpu.sync_copy(px_hbm, px_v)
        pltpu.sync_copy(py_hbm, py_v)
        pltpu.sync_copy(pz_hbm, pz_v)
        lane = lax.iota(jnp.int32, 16)
        bufs = (tjbuf0, tjbuf1)
        sems = (sem0, sem1)

        def start(j, b):
            ch0 = pl.multiple_of(j * CH, 8)
            pltpu.async_copy(t_hbm.at[src_v.at[pl.ds(ch0, CH)]], bufs[b],
                             sems[b])

        def finish(j, b):
            ch0 = pl.multiple_of(j * CH, 8)
            pltpu.make_async_copy(t_hbm.at[src_v.at[pl.ds(ch0, CH)]], bufs[b],
                                  sems[b]).wait()
            buf = bufs[b]
            for gi in range(CH // 16):
                off = pl.multiple_of(j * CH + gi * 16, 8)
                s16 = src_v[pl.ds(off, 16)]
                d16 = dst_v[pl.ds(off, 16)]
                dx = plsc.load_gather(px_v, [d16]) - plsc.load_gather(px_v, [s16])
                dy = plsc.load_gather(py_v, [d16]) - plsc.load_gather(py_v, [s16])
                dz = plsc.load_gather(pz_v, [d16]) - plsc.load_gather(pz_v, [s16])
                r2 = dx * dx + dy * dy + dz * dz
                for cix in range(16):
                    plsc.store_scatter(
                        buf, [gi * 16 + lane, jnp.full((16,), 64 + cix,
                                                       jnp.int32)], r2)
            pltpu.sync_copy(
                buf, tj_out.at[pl.ds(pl.multiple_of(base + j * CH, 8), CH)])

        start(0, 0)
        start(1, 1)

        @pl.loop(0, NCHK - 1, step=2)
        def _(j):
            finish(j, 0)

            @pl.when(j + 2 < NCHK)
            def _():
                start(j + 2, 0)

            finish(j + 1, 1)

            @pl.when(j + 3 < NCHK)
            def _():
                start(j + 3, 1)

        if NCHK % 2 == 1:
            finish(NCHK - 1, 0)

    return k(src, dst, T, px, py, pz)


def _edge_stage(TJ, edge_attr, Wsc, We, om2, bn1, Wn2, bn2, Wn3, bn3,
                blk_off=0):
    E = TJ.shape[0]
    C = edge_attr.shape[1]
    MID = Wn2.shape[0]
    BE = 2560
    grid = (E // BE,)

    def body(tj_ref, ea_ref, wsc_ref, we_ref, om_ref, bn1_ref,
             w2_ref, bn2_ref, w3_ref, bn3_ref, out_ref):
        tj = tj_ref[...]
        g = tj[:, :64]
        ang = jnp.sqrt(tj[:, 64:80] * om_ref[...])
        sn, cs = _sincos(ang)
        sc = jnp.concatenate([sn, cs], axis=1)
        p1 = (sc @ wsc_ref[...]
              + ea_ref[...] @ we_ref[...] + g + bn1_ref[...])
        q1 = p1 * lax.erf(p1) + p1
        p2 = q1 @ w2_ref[...] + bn2_ref[...]
        q2 = p2 * lax.erf(p2) + p2
        out_ref[...] = q2 @ w3_ref[...] + bn3_ref[...]

    return pl.pallas_call(
        body,
        grid=grid,
        in_specs=[
            pl.BlockSpec((BE, C), lambda i: (i, 0)),
            pl.BlockSpec((BE, C), lambda i: (i + blk_off, 0)),
            pl.BlockSpec((32, MID), lambda i: (0, 0)),
            pl.BlockSpec((C, MID), lambda i: (0, 0)),
            pl.BlockSpec((1, 16), lambda i: (0, 0)),
            pl.BlockSpec((1, MID), lambda i: (0, 0)),
            pl.BlockSpec((MID, MID), lambda i: (0, 0)),
            pl.BlockSpec((1, MID), lambda i: (0, 0)),
            pl.BlockSpec((MID, C), lambda i: (0, 0)),
            pl.BlockSpec((1, C), lambda i: (0, 0)),
        ],
        out_specs=pl.BlockSpec((BE, C), lambda i: (i, 0)),
        out_shape=jax.ShapeDtypeStruct((E, C), jnp.float32),
    )(TJ, edge_attr, Wsc, We, om2, bn1, Wn2, bn2, Wn3, bn3)


def _sc_scatter(dst3d, msg, zeros):
    E, C = msg.shape
    N = zeros.shape[0]
    EPW = E // NW
    NCHK = EPW // CH
    RPT = 640
    TAIL = N - RPT * (NS - 1)
    mesh = plsc.VectorSubcoreMesh(
        core_axis_name="c", subcore_axis_name="s", num_cores=NC, num_subcores=NS
    )

    @functools.partial(
        pl.kernel,
        mesh=mesh,
        compiler_params=pltpu.CompilerParams(needs_layout_passes=False),
        out_type=jax.ShapeDtypeStruct((NC * N, C), jnp.float32),
        scratch_types=[
            pltpu.VMEM((NCHK, CH), jnp.int32),
            pltpu.VMEM((CH, C), jnp.float32),
            pltpu.VMEM((CH, C), jnp.float32),
            pltpu.VMEM_SHARED((N, C), jnp.float32),
            pltpu.SemaphoreType.DMA,
            pltpu.SemaphoreType.DMA,
        ],
    )
    def k(dst3d_hbm, msg_hbm, z_hbm, agg_out, idx_v, mbuf0, mbuf1, agg_sh,
          sem0, sem1):
        c = lax.axis_index("c")
        s = lax.axis_index("s")
        w = c * NS + s

        @pl.when(s < NS - 1)
        def _():
            r0 = pl.multiple_of(s * RPT, 8)
            pltpu.sync_copy(z_hbm.at[pl.ds(r0, RPT)], agg_sh.at[pl.ds(r0, RPT)])

        @pl.when(s == NS - 1)
        def _():
            r0 = RPT * (NS - 1)
            pltpu.sync_copy(z_hbm.at[pl.ds(r0, TAIL)], agg_sh.at[pl.ds(r0, TAIL)])

        pltpu.sync_copy(dst3d_hbm.at[w], idx_v)
        plsc.subcore_barrier()
        bufs = (mbuf0, mbuf1)
        sems = (sem0, sem1)

        def start(j, b):
            e0 = pl.multiple_of(w * EPW + j * CH, 8)
            pltpu.async_copy(msg_hbm.at[pl.ds(e0, CH)], bufs[b], sems[b])

        def finish(j, b):
            e0 = pl.multiple_of(w * EPW + j * CH, 8)
            pltpu.make_async_copy(msg_hbm.at[pl.ds(e0, CH)], bufs[b],
                                  sems[b]).wait()
            pltpu.sync_copy(bufs[b], agg_sh.at[idx_v.at[j]], add=True)

        start(0, 0)
        start(1, 1)

        @pl.loop(0, NCHK - 1, step=2)
        def _(j):
            finish(j, 0)

            @pl.when(j + 2 < NCHK)
            def _():
                start(j + 2, 0)

            finish(j + 1, 1)

            @pl.when(j + 3 < NCHK)
            def _():
                start(j + 3, 1)

        if NCHK % 2 == 1:
            finish(NCHK - 1, 0)
        plsc.subcore_barrier()

        @pl.when(s < NS - 1)
        def _():
            r0 = pl.multiple_of(s * RPT, 8)
            o0 = pl.multiple_of(c * N + s * RPT, 8)
            pltpu.sync_copy(agg_sh.at[pl.ds(r0, RPT)], agg_out.at[pl.ds(o0, RPT)])

        @pl.when(s == NS - 1)
        def _():
            r0 = RPT * (NS - 1)
            o0 = pl.multiple_of(c * N + r0, 8)
            pltpu.sync_copy(agg_sh.at[pl.ds(r0, TAIL)], agg_out.at[pl.ds(o0, TAIL)])

    return k(dst3d, msg, zeros)


def _final_stage(h, aggpA, aggpB, W_a2, b_a2, W_a3, b_a3):
    N, C = h.shape
    BN = 1000
    grid = (N // BN,)

    def body(h_ref, a0_ref, a1_ref, a2_ref, a3_ref, wa2_ref, ba2_ref,
             wa3_ref, ba3_ref, out_ref):
        hb = (h_ref[...] + a0_ref[...] + a1_ref[...]
              + a2_ref[...] + a3_ref[...])
        t = jnp.maximum(hb @ wa2_ref[...] + ba2_ref[...], 0.0)
        out_ref[...] = t @ wa3_ref[...] + ba3_ref[...]

    nb = N // BN
    return pl.pallas_call(
        body,
        grid=grid,
        in_specs=[
            pl.BlockSpec((BN, C), lambda i: (i, 0)),
            pl.BlockSpec((BN, C), lambda i: (i, 0)),
            pl.BlockSpec((BN, C), lambda i: (i + nb, 0)),
            pl.BlockSpec((BN, C), lambda i: (i, 0)),
            pl.BlockSpec((BN, C), lambda i: (i + nb, 0)),
            pl.BlockSpec((C, C), lambda i: (0, 0)),
            pl.BlockSpec((1, C), lambda i: (0, 0)),
            pl.BlockSpec((C, C), lambda i: (0, 0)),
            pl.BlockSpec((1, C), lambda i: (0, 0)),
        ],
        out_specs=pl.BlockSpec((BN, C), lambda i: (i, 0)),
        out_shape=jax.ShapeDtypeStruct((N, C), jnp.float32),
    )(h, aggpA, aggpA, aggpB, aggpB, W_a2, b_a2, W_a3, b_a3)


def kernel(x, edge_index, edge_attr, x_pos, W_a1, b_a1, Wn1, bn1, Wn2, bn2,
           Wn3, bn3, W_a2, b_a2, W_a3, b_a3):
    N, C = x.shape
    E = edge_index.shape[1]
    NF = 16
    n_channels = 128
    omeg = jnp.asarray(
        [10.0 * (float(n_channels) ** (1.0 - 2.0 * i / NF)) for i in range(NF)],
        jnp.float32).reshape(1, NF)

    src = edge_index[0]
    dst = edge_index[1]
    rs2 = np.float32(0.7071067811865476)
    Wsc = Wn1[: 2 * NF] * rs2
    We = Wn1[2 * NF: 2 * NF + C] * rs2
    Wg = Wn1[2 * NF + C:] * rs2
    bn1 = bn1 * rs2
    Wn2 = Wn2 * np.float32(0.5)
    bn2 = bn2 * rs2
    Wn3 = Wn3 * rs2

    h, T = _node_stage(x, W_a1, b_a1.reshape(1, C), Wg)
    px, py, pz = x_pos[:, 0], x_pos[:, 1], x_pos[:, 2]
    om2 = omeg * omeg
    zer = jnp.zeros((N, C), jnp.float32)
    EA = (E // 2 // (NW * CH)) * (NW * CH)
    srcA, dstA = src[:EA], dst[:EA]
    srcB, dstB = src[EA:], dst[EA:]
    TJA = _sc_gather(srcA, dstA, T, px, py, pz)
    TJB = _sc_gather(srcB, dstB, T, px, py, pz)
    bn1r, bn2r, bn3r = bn1.reshape(1, -1), bn2.reshape(1, -1), bn3.reshape(1, C)
    msgA = _edge_stage(TJA, edge_attr, Wsc, We, om2, bn1r, Wn2, bn2r,
                       Wn3, bn3r, blk_off=0)
    msgB = _edge_stage(TJB, edge_attr, Wsc, We, om2, bn1r, Wn2, bn2r,
                       Wn3, bn3r, blk_off=EA // 2560)
    aggA = _sc_scatter(dstA.reshape(NW, EA // (NW * CH), CH), msgA, zer)
    aggB = _sc_scatter(dstB.reshape(NW, (E - EA) // (NW * CH), CH), msgB, zer)
    return _final_stage(h, aggA, aggB, W_a2, b_a2.reshape(1, C), W_a3,
                        b_a3.reshape(1, C))


# speedup vs baseline: 6.2738x; 1.0264x over previous
import functools

import jax
import jax.numpy as jnp
import numpy as np
from jax import lax
from jax.experimental import pallas as pl
from jax.experimental.pallas import tpu as pltpu
from jax.experimental.pallas import tpu_sc as plsc

NC = 2
NS = 16
NW = NC * NS
CH = 80


def _sincos(ang):
    nf = jnp.round(ang * np.float32(2.0 / np.pi))
    ni = nf.astype(jnp.int32)
    x = ((ang - nf * np.float32(1.5703125))
         - nf * np.float32(4.837512969970703125e-4)) \
        - nf * np.float32(7.54978995489188216e-8)
    z = x * x
    sp = ((np.float32(-1.9515295891e-4) * z + np.float32(8.3321608736e-3)) * z
          + np.float32(-1.6666654611e-1)) * z * x + x
    cp = ((np.float32(2.443315711809948e-5) * z
           + np.float32(-1.388731625493765e-3)) * z
          + np.float32(4.166664568298827e-2)) * z * z \
        - np.float32(0.5) * z + np.float32(1.0)
    q0 = (ni & 1) != 0
    ssel = jnp.where(q0, cp, sp)
    csel = jnp.where(q0, sp, cp)
    sflip = (ni & 2) << 30
    cflip = ((ni + 1) & 2) << 30
    s = lax.bitcast_convert_type(
        lax.bitcast_convert_type(ssel, jnp.int32) ^ sflip, jnp.float32)
    c = lax.bitcast_convert_type(
        lax.bitcast_convert_type(csel, jnp.int32) ^ cflip, jnp.float32)
    return s, c


def _node_stage(x, W_a1, b_a1, Wg):
    N, C = x.shape
    BN = 1000
    grid = (N // BN,)

    def body(x_ref, wa1_ref, ba1_ref, wg_ref, h_ref, t_ref):
        hb = x_ref[...] @ wa1_ref[...] + ba1_ref[...]
        h_ref[...] = hb
        g = hb @ wg_ref[...]
        t_ref[...] = jnp.concatenate([g, jnp.zeros_like(g)], axis=1)

    return pl.pallas_call(
        body,
        grid=grid,
        in_specs=[
            pl.BlockSpec((BN, C), lambda i: (i, 0)),
            pl.BlockSpec((C, C), lambda i: (0, 0)),
            pl.BlockSpec((1, C), lambda i: (0, 0)),
            pl.BlockSpec((C, 64), lambda i: (0, 0)),
        ],
        out_specs=[
            pl.BlockSpec((BN, C), lambda i: (i, 0)),
            pl.BlockSpec((BN, C), lambda i: (i, 0)),
        ],
        out_shape=[
            jax.ShapeDtypeStruct((N, C), jnp.float32),
            jax.ShapeDtypeStruct((N, C), jnp.float32),
        ],
    )(x, W_a1, b_a1, Wg)


def _sc_gather(src, dst, T, px, py, pz):
    E = src.shape[0]
    N, C = T.shape
    EPW = E // NW
    NCHK = EPW // CH
    mesh = plsc.VectorSubcoreMesh(
        core_axis_name="c", subcore_axis_name="s", num_cores=NC, num_subcores=NS
    )

    @functools.partial(
        pl.kernel,
        mesh=mesh,
        out_type=jax.ShapeDtypeStruct((E, C), jnp.float32),
        compiler_params=pltpu.CompilerParams(needs_layout_passes=False),
        scratch_types=[
            pltpu.VMEM((EPW,), jnp.int32),
            pltpu.VMEM((EPW,), jnp.int32),
            pltpu.VMEM((N,), jnp.float32),
            pltpu.VMEM((N,), jnp.float32),
            pltpu.VMEM((N,), jnp.float32),
            pltpu.VMEM((CH, C), jnp.float32),
            pltpu.VMEM((CH, C), jnp.float32),
            pltpu.SemaphoreType.DMA,
            pltpu.SemaphoreType.DMA,
        ],
    )
    def k(src_hbm, dst_hbm, t_hbm, px_hbm, py_hbm, pz_hbm, tj_out,
          src_v, dst_v, px_v, py_v, pz_v, tjbuf0, tjbuf1, sem0, sem1):
        w = lax.axis_index("c") * NS + lax.axis_index("s")
        base = pl.multiple_of(w * EPW, 8)
        pltpu.sync_copy(src_hbm.at[pl.ds(base, EPW)], src_v)
        pltpu.sync_copy(dst_hbm.at[pl.ds(base, EPW)], dst_v)
        pltpu.sync_copy(px_hbm, px_v)
        pltpu.sync_copy(py_hbm, py_v)
        pltpu.sync_copy(pz_hbm, pz_v)
        lane = lax.iota(jnp.int32, 16)
        bufs = (tjbuf0, tjbuf1)
        sems = (sem0, sem1)

        def start(j, b):
            ch0 = pl.multiple_of(j * CH, 8)
            pltpu.async_copy(t_hbm.at[src_v.at[pl.ds(ch0, CH)]], bufs[b],
                             sems[b])

        def finish(j, b):
            ch0 = pl.multiple_of(j * CH, 8)
            pltpu.make_async_copy(t_hbm.at[src_v.at[pl.ds(ch0, CH)]], bufs[b],
                                  sems[b]).wait()
            buf = bufs[b]
            for gi in range(CH // 16):
                off = pl.multiple_of(j * CH + gi * 16, 8)
                s16 = src_v[pl.ds(off, 16)]
                d16 = dst_v[pl.ds(off, 16)]
                dx = plsc.load_gather(px_v, [d16]) - plsc.load_gather(px_v, [s16])
                dy = plsc.load_gather(py_v, [d16]) - plsc.load_gather(py_v, [s16])
                dz = plsc.load_gather(pz_v, [d16]) - plsc.load_gather(pz_v, [s16])
                r2 = dx * dx + dy * dy + dz * dz
                for cix in range(16):
                    plsc.store_scatter(
                        buf, [gi * 16 + lane, jnp.full((16,), 64 + cix,
                                                       jnp.int32)], r2)
            pltpu.sync_copy(
                buf, tj_out.at[pl.ds(pl.multiple_of(base + j * CH, 8), CH)])

        start(0, 0)
        start(1, 1)

        @pl.loop(0, NCHK - 1, step=2)
        def _(j):
            finish(j, 0)

            @pl.when(j + 2 < NCHK)
            def _():
                start(j + 2, 0)

            finish(j + 1, 1)

            @pl.when(j + 3 < NCHK)
            def _():
                start(j + 3, 1)

        if NCHK % 2 == 1:
            finish(NCHK - 1, 0)

    return k(src, dst, T, px, py, pz)


def _edge_stage(TJ, edge_attr, Wsc, We, om2, bn1, Wn2, bn2, Wn3, bn3,
                blk_off=0):
    E = TJ.shape[0]
    C = edge_attr.shape[1]
    MID = Wn2.shape[0]
    BE = 2560
    grid = (E // BE,)

    def body(tj_ref, ea_ref, wsc_ref, we_ref, om_ref, bn1_ref,
             w2_ref, bn2_ref, w3_ref, bn3_ref, out_ref):
        tj = tj_ref[...]
        g = tj[:, :64]
        ang = jnp.sqrt(tj[:, 64:80] * om_ref[...])
        sn, cs = _sincos(ang)
        sc = jnp.concatenate([sn, cs], axis=1)
        p1 = (sc @ wsc_ref[...]
              + ea_ref[...] @ we_ref[...] + g + bn1_ref[...])
        q1 = p1 * lax.erf(p1) + p1
        p2 = q1 @ w2_ref[...] + bn2_ref[...]
        q2 = p2 * lax.erf(p2) + p2
        out_ref[...] = q2 @ w3_ref[...] + bn3_ref[...]

    return pl.pallas_call(
        body,
        grid=grid,
        in_specs=[
            pl.BlockSpec((BE, C), lambda i: (i, 0)),
            pl.BlockSpec((BE, C), lambda i: (i + blk_off, 0)),
            pl.BlockSpec((32, MID), lambda i: (0, 0)),
            pl.BlockSpec((C, MID), lambda i: (0, 0)),
            pl.BlockSpec((1, 16), lambda i: (0, 0)),
            pl.BlockSpec((1, MID), lambda i: (0, 0)),
            pl.BlockSpec((MID, MID), lambda i: (0, 0)),
            pl.BlockSpec((1, MID), lambda i: (0, 0)),
            pl.BlockSpec((MID, C), lambda i: (0, 0)),
            pl.BlockSpec((1, C), lambda i: (0, 0)),
        ],
        out_specs=pl.BlockSpec((BE, C), lambda i: (i, 0)),
        out_shape=jax.ShapeDtypeStruct((E, C), jnp.float32),
    )(TJ, edge_attr, Wsc, We, om2, bn1, Wn2, bn2, Wn3, bn3)


def _sc_scatter(dst3d, msg, zeros):
    E, C = msg.shape
    N = zeros.shape[0]
    EPW = E // NW
    NCHK = EPW // CH
    RPT = 640
    TAIL = N - RPT * (NS - 1)
    mesh = plsc.VectorSubcoreMesh(
        core_axis_name="c", subcore_axis_name="s", num_cores=NC, num_subcores=NS
    )

    @functools.partial(
        pl.kernel,
        mesh=mesh,
        compiler_params=pltpu.CompilerParams(needs_layout_passes=False),
        out_type=jax.ShapeDtypeStruct((NC * N, C), jnp.float32),
        scratch_types=[
            pltpu.VMEM((NCHK, CH), jnp.int32),
            pltpu.VMEM((CH, C), jnp.float32),
            pltpu.VMEM((CH, C), jnp.float32),
            pltpu.VMEM_SHARED((N, C), jnp.float32),
            pltpu.SemaphoreType.DMA,
            pltpu.SemaphoreType.DMA,
        ],
    )
    def k(dst3d_hbm, msg_hbm, z_hbm, agg_out, idx_v, mbuf0, mbuf1, agg_sh,
          sem0, sem1):
        c = lax.axis_index("c")
        s = lax.axis_index("s")
        w = c * NS + s

        @pl.when(s < NS - 1)
        def _():
            r0 = pl.multiple_of(s * RPT, 8)
            pltpu.sync_copy(z_hbm.at[pl.ds(r0, RPT)], agg_sh.at[pl.ds(r0, RPT)])

        @pl.when(s == NS - 1)
        def _():
            r0 = RPT * (NS - 1)
            pltpu.sync_copy(z_hbm.at[pl.ds(r0, TAIL)], agg_sh.at[pl.ds(r0, TAIL)])

        pltpu.sync_copy(dst3d_hbm.at[w], idx_v)
        plsc.subcore_barrier()
        bufs = (mbuf0, mbuf1)
        sems = (sem0, sem1)

        def start(j, b):
            e0 = pl.multiple_of(w * EPW + j * CH, 8)
            pltpu.async_copy(msg_hbm.at[pl.ds(e0, CH)], bufs[b], sems[b])

        def finish(j, b):
            e0 = pl.multiple_of(w * EPW + j * CH, 8)
            pltpu.make_async_copy(msg_hbm.at[pl.ds(e0, CH)], bufs[b],
                                  sems[b]).wait()
            pltpu.sync_copy(bufs[b], agg_sh.at[idx_v.at[j]], add=True)

        start(0, 0)
        start(1, 1)

        @pl.loop(0, NCHK - 1, step=2)
        def _(j):
            finish(j, 0)

            @pl.when(j + 2 < NCHK)
            def _():
                start(j + 2, 0)

            finish(j + 1, 1)

            @pl.when(j + 3 < NCHK)
            def _():
                start(j + 3, 1)

        if NCHK % 2 == 1:
            finish(NCHK - 1, 0)
        plsc.subcore_barrier()

        @pl.when(s < NS - 1)
        def _():
            r0 = pl.multiple_of(s * RPT, 8)
            o0 = pl.multiple_of(c * N + s * RPT, 8)
            pltpu.sync_copy(agg_sh.at[pl.ds(r0, RPT)], agg_out.at[pl.ds(o0, RPT)])

        @pl.when(s == NS - 1)
        def _():
            r0 = RPT * (NS - 1)
            o0 = pl.multiple_of(c * N + r0, 8)
            pltpu.sync_copy(agg_sh.at[pl.ds(r0, TAIL)], agg_out.at[pl.ds(o0, TAIL)])

    return k(dst3d, msg, zeros)


def _final_stage(h, aggps, W_a2, b_a2, W_a3, b_a3):
    N, C = h.shape
    BN = 1000
    grid = (N // BN,)

    nparts = len(aggps)

    def body(*refs):
        h_ref = refs[0]
        aggs = refs[1:1 + 2 * nparts]
        wa2_ref, ba2_ref, wa3_ref, ba3_ref, out_ref = refs[1 + 2 * nparts:]
        hb = h_ref[...]
        for a in aggs:
            hb = hb + a[...]
        t = jnp.maximum(hb @ wa2_ref[...] + ba2_ref[...], 0.0)
        out_ref[...] = t @ wa3_ref[...] + ba3_ref[...]

    nb = N // BN
    agg_specs = []
    agg_args = []
    for a in aggps:
        agg_specs.append(pl.BlockSpec((BN, C), lambda i: (i, 0)))
        agg_specs.append(pl.BlockSpec((BN, C), lambda i: (i + nb, 0)))
        agg_args.extend([a, a])
    return pl.pallas_call(
        body,
        grid=grid,
        in_specs=[pl.BlockSpec((BN, C), lambda i: (i, 0))] + agg_specs + [
            pl.BlockSpec((C, C), lambda i: (0, 0)),
            pl.BlockSpec((1, C), lambda i: (0, 0)),
            pl.BlockSpec((C, C), lambda i: (0, 0)),
            pl.BlockSpec((1, C), lambda i: (0, 0)),
        ],
        out_specs=pl.BlockSpec((BN, C), lambda i: (i, 0)),
        out_shape=jax.ShapeDtypeStruct((N, C), jnp.float32),
    )(h, *agg_args, W_a2, b_a2, W_a3, b_a3)


def kernel(x, edge_index, edge_attr, x_pos, W_a1, b_a1, Wn1, bn1, Wn2, bn2,
           Wn3, bn3, W_a2, b_a2, W_a3, b_a3):
    N, C = x.shape
    E = edge_index.shape[1]
    NF = 16
    n_channels = 128
    omeg = jnp.asarray(
        [10.0 * (float(n_channels) ** (1.0 - 2.0 * i / NF)) for i in range(NF)],
        jnp.float32).reshape(1, NF)

    src = edge_index[0]
    dst = edge_index[1]
    rs2 = np.float32(0.7071067811865476)
    Wsc = Wn1[: 2 * NF] * rs2
    We = Wn1[2 * NF: 2 * NF + C] * rs2
    Wg = Wn1[2 * NF + C:] * rs2
    bn1 = bn1 * rs2
    Wn2 = Wn2 * np.float32(0.5)
    bn2 = bn2 * rs2
    Wn3 = Wn3 * rs2

    h, T = _node_stage(x, W_a1, b_a1.reshape(1, C), Wg)
    px, py, pz = x_pos[:, 0], x_pos[:, 1], x_pos[:, 2]
    om2 = omeg * omeg
    zer = jnp.zeros((N, C), jnp.float32)
    G = NW * CH
    nblk = E // G
    if nblk >= 8:
        b1 = (nblk * 35) // 100
        b2 = (nblk * 37) // 100
        blks = [b1, b2, nblk - b1 - b2]
    else:
        blks = [nblk]
    bn1r, bn2r, bn3r = bn1.reshape(1, -1), bn2.reshape(1, -1), bn3.reshape(1, C)
    aggs = []
    e0 = 0
    for nb_i in blks:
        ei = nb_i * G
        srcP, dstP = src[e0:e0 + ei], dst[e0:e0 + ei]
        TJP = _sc_gather(srcP, dstP, T, px, py, pz)
        msgP = _edge_stage(TJP, edge_attr, Wsc, We, om2, bn1r, Wn2, bn2r,
                           Wn3, bn3r, blk_off=e0 // G)
        aggs.append(_sc_scatter(dstP.reshape(NW, ei // G, CH), msgP, zer))
        e0 += ei
    return _final_stage(h, aggs, W_a2, b_a2.reshape(1, C), W_a3,
                        b_a3.reshape(1, C))


# speedup vs baseline: 6.2855x; 1.0019x over previous
import functools

import jax
import jax.numpy as jnp
import numpy as np
from jax import lax
from jax.experimental import pallas as pl
from jax.experimental.pallas import tpu as pltpu
from jax.experimental.pallas import tpu_sc as plsc

NC = 2
NS = 16
NW = NC * NS
CH = 80


def _sincos(ang):
    nf = jnp.round(ang * np.float32(2.0 / np.pi))
    ni = nf.astype(jnp.int32)
    x = ((ang - nf * np.float32(1.5703125))
         - nf * np.float32(4.837512969970703125e-4)) \
        - nf * np.float32(7.54978995489188216e-8)
    z = x * x
    sp = ((np.float32(-1.9515295891e-4) * z + np.float32(8.3321608736e-3)) * z
          + np.float32(-1.6666654611e-1)) * z * x + x
    cp = ((np.float32(2.443315711809948e-5) * z
           + np.float32(-1.388731625493765e-3)) * z
          + np.float32(4.166664568298827e-2)) * z * z \
        - np.float32(0.5) * z + np.float32(1.0)
    q0 = (ni & 1) != 0
    ssel = jnp.where(q0, cp, sp)
    csel = jnp.where(q0, sp, cp)
    sflip = (ni & 2) << 30
    cflip = ((ni + 1) & 2) << 30
    s = lax.bitcast_convert_type(
        lax.bitcast_convert_type(ssel, jnp.int32) ^ sflip, jnp.float32)
    c = lax.bitcast_convert_type(
        lax.bitcast_convert_type(csel, jnp.int32) ^ cflip, jnp.float32)
    return s, c


def _node_stage(x, W_a1, b_a1, Wg):
    N, C = x.shape
    BN = 1000
    grid = (N // BN,)

    def body(x_ref, wa1_ref, ba1_ref, wg_ref, h_ref, t_ref):
        hb = x_ref[...] @ wa1_ref[...] + ba1_ref[...]
        h_ref[...] = hb
        g = hb @ wg_ref[...]
        t_ref[...] = jnp.concatenate([g, jnp.zeros_like(g)], axis=1)

    return pl.pallas_call(
        body,
        grid=grid,
        in_specs=[
            pl.BlockSpec((BN, C), lambda i: (i, 0)),
            pl.BlockSpec((C, C), lambda i: (0, 0)),
            pl.BlockSpec((1, C), lambda i: (0, 0)),
            pl.BlockSpec((C, 64), lambda i: (0, 0)),
        ],
        out_specs=[
            pl.BlockSpec((BN, C), lambda i: (i, 0)),
            pl.BlockSpec((BN, C), lambda i: (i, 0)),
        ],
        out_shape=[
            jax.ShapeDtypeStruct((N, C), jnp.float32),
            jax.ShapeDtypeStruct((N, C), jnp.float32),
        ],
    )(x, W_a1, b_a1, Wg)


def _sc_gather(src, dst, T, px, py, pz):
    E = src.shape[0]
    N, C = T.shape
    EPW = E // NW
    NCHK = EPW // CH
    mesh = plsc.VectorSubcoreMesh(
        core_axis_name="c", subcore_axis_name="s", num_cores=NC, num_subcores=NS
    )

    @functools.partial(
        pl.kernel,
        mesh=mesh,
        out_type=jax.ShapeDtypeStruct((E, C), jnp.float32),
        compiler_params=pltpu.CompilerParams(needs_layout_passes=False),
        scratch_types=[
            pltpu.VMEM((EPW,), jnp.int32),
            pltpu.VMEM((EPW,), jnp.int32),
            pltpu.VMEM((N,), jnp.float32),
            pltpu.VMEM((N,), jnp.float32),
            pltpu.VMEM((N,), jnp.float32),
            pltpu.VMEM((CH, C), jnp.float32),
            pltpu.VMEM((CH, C), jnp.float32),
            pltpu.SemaphoreType.DMA,
            pltpu.SemaphoreType.DMA,
        ],
    )
    def k(src_hbm, dst_hbm, t_hbm, px_hbm, py_hbm, pz_hbm, tj_out,
          src_v, dst_v, px_v, py_v, pz_v, tjbuf0, tjbuf1, sem0, sem1):
        w = lax.axis_index("c") * NS + lax.axis_index("s")
        base = pl.multiple_of(w * EPW, 8)
        c1 = pltpu.async_copy(src_hbm.at[pl.ds(base, EPW)], src_v, sem0)
        c2 = pltpu.async_copy(dst_hbm.at[pl.ds(base, EPW)], dst_v, sem0)
        c3 = pltpu.async_copy(px_hbm, px_v, sem1)
        c4 = pltpu.async_copy(py_hbm, py_v, sem1)
        c5 = pltpu.async_copy(pz_hbm, pz_v, sem1)
        c1.wait()
        c2.wait()
        c3.wait()
        c4.wait()
        c5.wait()
        lane = lax.iota(jnp.int32, 16)
        bufs = (tjbuf0, tjbuf1)
        sems = (sem0, sem1)

        def start(j, b):
            ch0 = pl.multiple_of(j * CH, 8)
            pltpu.async_copy(t_hbm.at[src_v.at[pl.ds(ch0, CH)]], bufs[b],
                             sems[b])

        def finish(j, b):
            ch0 = pl.multiple_of(j * CH, 8)
            pltpu.make_async_copy(t_hbm.at[src_v.at[pl.ds(ch0, CH)]], bufs[b],
                                  sems[b]).wait()
            buf = bufs[b]
            for gi in range(CH // 16):
                off = pl.multiple_of(j * CH + gi * 16, 8)
                s16 = src_v[pl.ds(off, 16)]
                d16 = dst_v[pl.ds(off, 16)]
                dx = plsc.load_gather(px_v, [d16]) - plsc.load_gather(px_v, [s16])
                dy = plsc.load_gather(py_v, [d16]) - plsc.load_gather(py_v, [s16])
                dz = plsc.load_gather(pz_v, [d16]) - plsc.load_gather(pz_v, [s16])
                r2 = dx * dx + dy * dy + dz * dz
                for cix in range(16):
                    plsc.store_scatter(
                        buf, [gi * 16 + lane, jnp.full((16,), 64 + cix,
                                                       jnp.int32)], r2)
            pltpu.sync_copy(
                buf, tj_out.at[pl.ds(pl.multiple_of(base + j * CH, 8), CH)])

        start(0, 0)
        start(1, 1)

        @pl.loop(0, NCHK - 1, step=2)
        def _(j):
            finish(j, 0)

            @pl.when(j + 2 < NCHK)
            def _():
                start(j + 2, 0)

            finish(j + 1, 1)

            @pl.when(j + 3 < NCHK)
            def _():
                start(j + 3, 1)

        if NCHK % 2 == 1:
            finish(NCHK - 1, 0)

    return k(src, dst, T, px, py, pz)


def _edge_stage(TJ, edge_attr, Wsc, We, om2, bn1, Wn2, bn2, Wn3, bn3,
                blk_off=0):
    E = TJ.shape[0]
    C = edge_attr.shape[1]
    MID = Wn2.shape[0]
    BE = 2560
    grid = (E // BE,)

    def body(tj_ref, ea_ref, wsc_ref, we_ref, om_ref, bn1_ref,
             w2_ref, bn2_ref, w3_ref, bn3_ref, out_ref):
        tj = tj_ref[...]
        g = tj[:, :64]
        ang = jnp.sqrt(tj[:, 64:80] * om_ref[...])
        sn, cs = _sincos(ang)
        sc = jnp.concatenate([sn, cs], axis=1)
        p1 = (sc @ wsc_ref[...]
              + ea_ref[...] @ we_ref[...] + g + bn1_ref[...])
        q1 = p1 * lax.erf(p1) + p1
        p2 = q1 @ w2_ref[...] + bn2_ref[...]
        q2 = p2 * lax.erf(p2) + p2
        out_ref[...] = q2 @ w3_ref[...] + bn3_ref[...]

    return pl.pallas_call(
        body,
        grid=grid,
        in_specs=[
            pl.BlockSpec((BE, C), lambda i: (i, 0)),
            pl.BlockSpec((BE, C), lambda i: (i + blk_off, 0)),
            pl.BlockSpec((32, MID), lambda i: (0, 0)),
            pl.BlockSpec((C, MID), lambda i: (0, 0)),
            pl.BlockSpec((1, 16), lambda i: (0, 0)),
            pl.BlockSpec((1, MID), lambda i: (0, 0)),
            pl.BlockSpec((MID, MID), lambda i: (0, 0)),
            pl.BlockSpec((1, MID), lambda i: (0, 0)),
            pl.BlockSpec((MID, C), lambda i: (0, 0)),
            pl.BlockSpec((1, C), lambda i: (0, 0)),
        ],
        out_specs=pl.BlockSpec((BE, C), lambda i: (i, 0)),
        out_shape=jax.ShapeDtypeStruct((E, C), jnp.float32),
    )(TJ, edge_attr, Wsc, We, om2, bn1, Wn2, bn2, Wn3, bn3)


def _sc_scatter(dst3d, msg, zeros):
    E, C = msg.shape
    N = zeros.shape[0]
    EPW = E // NW
    NCHK = EPW // CH
    RPT = 640
    TAIL = N - RPT * (NS - 1)
    mesh = plsc.VectorSubcoreMesh(
        core_axis_name="c", subcore_axis_name="s", num_cores=NC, num_subcores=NS
    )

    @functools.partial(
        pl.kernel,
        mesh=mesh,
        compiler_params=pltpu.CompilerParams(needs_layout_passes=False),
        out_type=jax.ShapeDtypeStruct((NC * N, C), jnp.float32),
        scratch_types=[
            pltpu.VMEM((NCHK, CH), jnp.int32),
            pltpu.VMEM((CH, C), jnp.float32),
            pltpu.VMEM((CH, C), jnp.float32),
            pltpu.VMEM_SHARED((N, C), jnp.float32),
            pltpu.SemaphoreType.DMA,
            pltpu.SemaphoreType.DMA,
        ],
    )
    def k(dst3d_hbm, msg_hbm, z_hbm, agg_out, idx_v, mbuf0, mbuf1, agg_sh,
          sem0, sem1):
        c = lax.axis_index("c")
        s = lax.axis_index("s")
        w = c * NS + s
        cidx = pltpu.async_copy(dst3d_hbm.at[w], idx_v, sem0)

        @pl.when(s < NS - 1)
        def _():
            r0 = pl.multiple_of(s * RPT, 8)
            pltpu.sync_copy(z_hbm.at[pl.ds(r0, RPT)], agg_sh.at[pl.ds(r0, RPT)])

        @pl.when(s == NS - 1)
        def _():
            r0 = RPT * (NS - 1)
            pltpu.sync_copy(z_hbm.at[pl.ds(r0, TAIL)], agg_sh.at[pl.ds(r0, TAIL)])

        cidx.wait()
        plsc.subcore_barrier()
        bufs = (mbuf0, mbuf1)
        sems = (sem0, sem1)

        def start(j, b):
            e0 = pl.multiple_of(w * EPW + j * CH, 8)
            pltpu.async_copy(msg_hbm.at[pl.ds(e0, CH)], bufs[b], sems[b])

        def finish(j, b):
            e0 = pl.multiple_of(w * EPW + j * CH, 8)
            pltpu.make_async_copy(msg_hbm.at[pl.ds(e0, CH)], bufs[b],
                                  sems[b]).wait()
            pltpu.sync_copy(bufs[b], agg_sh.at[idx_v.at[j]], add=True)

        start(0, 0)
        start(1, 1)

        @pl.loop(0, NCHK - 1, step=2)
        def _(j):
            finish(j, 0)

            @pl.when(j + 2 < NCHK)
            def _():
                start(j + 2, 0)

            finish(j + 1, 1)

            @pl.when(j + 3 < NCHK)
            def _():
                start(j + 3, 1)

        if NCHK % 2 == 1:
            finish(NCHK - 1, 0)
        plsc.subcore_barrier()

        @pl.when(s < NS - 1)
        def _():
            r0 = pl.multiple_of(s * RPT, 8)
            o0 = pl.multiple_of(c * N + s * RPT, 8)
            pltpu.sync_copy(agg_sh.at[pl.ds(r0, RPT)], agg_out.at[pl.ds(o0, RPT)])

        @pl.when(s == NS - 1)
        def _():
            r0 = RPT * (NS - 1)
            o0 = pl.multiple_of(c * N + r0, 8)
            pltpu.sync_copy(agg_sh.at[pl.ds(r0, TAIL)], agg_out.at[pl.ds(o0, TAIL)])

    return k(dst3d, msg, zeros)


def _final_stage(h, aggps, W_a2, b_a2, W_a3, b_a3):
    N, C = h.shape
    BN = 1000
    grid = (N // BN,)

    nparts = len(aggps)

    def body(*refs):
        h_ref = refs[0]
        aggs = refs[1:1 + 2 * nparts]
        wa2_ref, ba2_ref, wa3_ref, ba3_ref, out_ref = refs[1 + 2 * nparts:]
        hb = h_ref[...]
        for a in aggs:
            hb = hb + a[...]
        t = jnp.maximum(hb @ wa2_ref[...] + ba2_ref[...], 0.0)
        out_ref[...] = t @ wa3_ref[...] + ba3_ref[...]

    nb = N // BN
    agg_specs = []
    agg_args = []
    for a in aggps:
        agg_specs.append(pl.BlockSpec((BN, C), lambda i: (i, 0)))
        agg_specs.append(pl.BlockSpec((BN, C), lambda i: (i + nb, 0)))
        agg_args.extend([a, a])
    return pl.pallas_call(
        body,
        grid=grid,
        in_specs=[pl.BlockSpec((BN, C), lambda i: (i, 0))] + agg_specs + [
            pl.BlockSpec((C, C), lambda i: (0, 0)),
            pl.BlockSpec((1, C), lambda i: (0, 0)),
            pl.BlockSpec((C, C), lambda i: (0, 0)),
            pl.BlockSpec((1, C), lambda i: (0, 0)),
        ],
        out_specs=pl.BlockSpec((BN, C), lambda i: (i, 0)),
        out_shape=jax.ShapeDtypeStruct((N, C), jnp.float32),
    )(h, *agg_args, W_a2, b_a2, W_a3, b_a3)


def kernel(x, edge_index, edge_attr, x_pos, W_a1, b_a1, Wn1, bn1, Wn2, bn2,
           Wn3, bn3, W_a2, b_a2, W_a3, b_a3):
    N, C = x.shape
    E = edge_index.shape[1]
    NF = 16
    n_channels = 128
    omeg = jnp.asarray(
        [10.0 * (float(n_channels) ** (1.0 - 2.0 * i / NF)) for i in range(NF)],
        jnp.float32).reshape(1, NF)

    src = edge_index[0]
    dst = edge_index[1]
    rs2 = np.float32(0.7071067811865476)
    Wsc = Wn1[: 2 * NF] * rs2
    We = Wn1[2 * NF: 2 * NF + C] * rs2
    Wg = Wn1[2 * NF + C:] * rs2
    bn1 = bn1 * rs2
    Wn2 = Wn2 * np.float32(0.5)
    bn2 = bn2 * rs2
    Wn3 = Wn3 * rs2

    h, T = _node_stage(x, W_a1, b_a1.reshape(1, C), Wg)
    px, py, pz = x_pos[:, 0], x_pos[:, 1], x_pos[:, 2]
    om2 = omeg * omeg
    zer = jnp.zeros((N, C), jnp.float32)
    G = NW * CH
    nblk = E // G
    if nblk >= 8:
        b1 = (nblk * 35) // 100
        b2 = (nblk * 37) // 100
        blks = [b1, b2, nblk - b1 - b2]
    else:
        blks = [nblk]
    bn1r, bn2r, bn3r = bn1.reshape(1, -1), bn2.reshape(1, -1), bn3.reshape(1, C)
    aggs = []
    e0 = 0
    for nb_i in blks:
        ei = nb_i * G
        srcP, dstP = src[e0:e0 + ei], dst[e0:e0 + ei]
        TJP = _sc_gather(srcP, dstP, T, px, py, pz)
        msgP = _edge_stage(TJP, edge_attr, Wsc, We, om2, bn1r, Wn2, bn2r,
                           Wn3, bn3r, blk_off=e0 // G)
        aggs.append(_sc_scatter(dstP.reshape(NW, ei // G, CH), msgP, zer))
        e0 += ei
    return _final_stage(h, aggs, W_a2, b_a2.reshape(1, C), W_a3,
                        b_a3.reshape(1, C))


# speedup vs baseline: 6.4976x; 1.0337x over previous
import functools

import jax
import jax.numpy as jnp
import numpy as np
from jax import lax
from jax.experimental import pallas as pl
from jax.experimental.pallas import tpu as pltpu
from jax.experimental.pallas import tpu_sc as plsc

NC = 2
NS = 16
NW = NC * NS
CH = 80


def _sincos(ang):
    nf = jnp.round(ang * np.float32(2.0 / np.pi))
    ni = nf.astype(jnp.int32)
    x = ((ang - nf * np.float32(1.5703125))
         - nf * np.float32(4.837512969970703125e-4)) \
        - nf * np.float32(7.54978995489188216e-8)
    z = x * x
    sp = ((np.float32(-1.9515295891e-4) * z + np.float32(8.3321608736e-3)) * z
          + np.float32(-1.6666654611e-1)) * z * x + x
    cp = ((np.float32(2.443315711809948e-5) * z
           + np.float32(-1.388731625493765e-3)) * z
          + np.float32(4.166664568298827e-2)) * z * z \
        - np.float32(0.5) * z + np.float32(1.0)
    q0 = (ni & 1) != 0
    ssel = jnp.where(q0, cp, sp)
    csel = jnp.where(q0, sp, cp)
    sflip = (ni & 2) << 30
    cflip = ((ni + 1) & 2) << 30
    s = lax.bitcast_convert_type(
        lax.bitcast_convert_type(ssel, jnp.int32) ^ sflip, jnp.float32)
    c = lax.bitcast_convert_type(
        lax.bitcast_convert_type(csel, jnp.int32) ^ cflip, jnp.float32)
    return s, c


def _node_stage(x, W_a1, b_a1, Wg):
    N, C = x.shape
    BN = 1000
    grid = (N // BN,)

    def body(x_ref, wa1_ref, ba1_ref, wg_ref, h_ref, t_ref):
        hb = x_ref[...] @ wa1_ref[...] + ba1_ref[...]
        h_ref[...] = hb
        g = hb @ wg_ref[...]
        t_ref[...] = jnp.concatenate([g, jnp.zeros_like(g)], axis=1)

    return pl.pallas_call(
        body,
        grid=grid,
        in_specs=[
            pl.BlockSpec((BN, C), lambda i: (i, 0)),
            pl.BlockSpec((C, C), lambda i: (0, 0)),
            pl.BlockSpec((1, C), lambda i: (0, 0)),
            pl.BlockSpec((C, 64), lambda i: (0, 0)),
        ],
        out_specs=[
            pl.BlockSpec((BN, C), lambda i: (i, 0)),
            pl.BlockSpec((BN, C), lambda i: (i, 0)),
        ],
        out_shape=[
            jax.ShapeDtypeStruct((N, C), jnp.float32),
            jax.ShapeDtypeStruct((N, C), jnp.float32),
        ],
    )(x, W_a1, b_a1, Wg)


def _sc_gather(src, dst, T, px, py, pz):
    E = src.shape[0]
    N, C = T.shape
    EPW = E // NW
    NCHK = EPW // CH
    mesh = plsc.VectorSubcoreMesh(
        core_axis_name="c", subcore_axis_name="s", num_cores=NC, num_subcores=NS
    )

    @functools.partial(
        pl.kernel,
        mesh=mesh,
        out_type=jax.ShapeDtypeStruct((E, C), jnp.float32),
        compiler_params=pltpu.CompilerParams(needs_layout_passes=False),
        scratch_types=[
            pltpu.VMEM((EPW,), jnp.int32),
            pltpu.VMEM((EPW,), jnp.int32),
            pltpu.VMEM((N,), jnp.float32),
            pltpu.VMEM((N,), jnp.float32),
            pltpu.VMEM((N,), jnp.float32),
            pltpu.VMEM((CH, C), jnp.float32),
            pltpu.VMEM((CH, C), jnp.float32),
            pltpu.SemaphoreType.DMA,
            pltpu.SemaphoreType.DMA,
        ],
    )
    def k(src_hbm, dst_hbm, t_hbm, px_hbm, py_hbm, pz_hbm, tj_out,
          src_v, dst_v, px_v, py_v, pz_v, tjbuf0, tjbuf1, sem0, sem1):
        w = lax.axis_index("c") * NS + lax.axis_index("s")
        base = pl.multiple_of(w * EPW, 8)
        c1 = pltpu.async_copy(src_hbm.at[pl.ds(base, EPW)], src_v, sem0)
        c2 = pltpu.async_copy(dst_hbm.at[pl.ds(base, EPW)], dst_v, sem0)
        c3 = pltpu.async_copy(px_hbm, px_v, sem1)
        c4 = pltpu.async_copy(py_hbm, py_v, sem1)
        c5 = pltpu.async_copy(pz_hbm, pz_v, sem1)
        c1.wait()
        c2.wait()
        c3.wait()
        c4.wait()
        c5.wait()
        lane = lax.iota(jnp.int32, 16)
        bufs = (tjbuf0, tjbuf1)
        sems = (sem0, sem1)

        def start(j, b):
            ch0 = pl.multiple_of(j * CH, 8)
            pltpu.async_copy(t_hbm.at[src_v.at[pl.ds(ch0, CH)]], bufs[b],
                             sems[b])

        def finish(j, b):
            ch0 = pl.multiple_of(j * CH, 8)
            pltpu.make_async_copy(t_hbm.at[src_v.at[pl.ds(ch0, CH)]], bufs[b],
                                  sems[b]).wait()
            buf = bufs[b]
            for gi in range(CH // 16):
                off = pl.multiple_of(j * CH + gi * 16, 8)
                s16 = src_v[pl.ds(off, 16)]
                d16 = dst_v[pl.ds(off, 16)]
                dx = plsc.load_gather(px_v, [d16]) - plsc.load_gather(px_v, [s16])
                dy = plsc.load_gather(py_v, [d16]) - plsc.load_gather(py_v, [s16])
                dz = plsc.load_gather(pz_v, [d16]) - plsc.load_gather(pz_v, [s16])
                r2 = dx * dx + dy * dy + dz * dz
                for cix in range(16):
                    plsc.store_scatter(
                        buf, [gi * 16 + lane, jnp.full((16,), 64 + cix,
                                                       jnp.int32)], r2)
            pltpu.sync_copy(
                buf, tj_out.at[pl.ds(pl.multiple_of(base + j * CH, 8), CH)])

        start(0, 0)
        start(1, 1)

        @pl.loop(0, NCHK - 1, step=2)
        def _(j):
            finish(j, 0)

            @pl.when(j + 2 < NCHK)
            def _():
                start(j + 2, 0)

            finish(j + 1, 1)

            @pl.when(j + 3 < NCHK)
            def _():
                start(j + 3, 1)

        if NCHK % 2 == 1:
            finish(NCHK - 1, 0)

    return k(src, dst, T, px, py, pz)


def _edge_stage(TJ, edge_attr, Wsc, We, om2, bn1, Wn2, bn2, Wn3, bn3,
                blk_off=0):
    E = TJ.shape[0]
    C = edge_attr.shape[1]
    MID = Wn2.shape[0]
    BE = 2560
    grid = (E // BE,)

    def body(tj_ref, ea_ref, wsc_ref, we_ref, om_ref, bn1_ref,
             w2_ref, bn2_ref, w3_ref, bn3_ref, out_ref):
        tj = tj_ref[...]
        g = tj[:, :64]
        ang = jnp.sqrt(tj[:, 64:80] * om_ref[...])
        sn, cs = _sincos(ang)
        sc = jnp.concatenate([sn, cs], axis=1)
        p1 = (sc @ wsc_ref[...]
              + ea_ref[...] @ we_ref[...] + g + bn1_ref[...])
        q1 = p1 * lax.erf(p1) + p1
        p2 = q1 @ w2_ref[...] + bn2_ref[...]
        q2 = p2 * lax.erf(p2) + p2
        out_ref[...] = q2 @ w3_ref[...] + bn3_ref[...]

    return pl.pallas_call(
        body,
        grid=grid,
        in_specs=[
            pl.BlockSpec((BE, C), lambda i: (i, 0)),
            pl.BlockSpec((BE, C), lambda i: (i + blk_off, 0)),
            pl.BlockSpec((32, MID), lambda i: (0, 0)),
            pl.BlockSpec((C, MID), lambda i: (0, 0)),
            pl.BlockSpec((1, 16), lambda i: (0, 0)),
            pl.BlockSpec((1, MID), lambda i: (0, 0)),
            pl.BlockSpec((MID, MID), lambda i: (0, 0)),
            pl.BlockSpec((1, MID), lambda i: (0, 0)),
            pl.BlockSpec((MID, C), lambda i: (0, 0)),
            pl.BlockSpec((1, C), lambda i: (0, 0)),
        ],
        out_specs=pl.BlockSpec((BE, C), lambda i: (i, 0)),
        out_shape=jax.ShapeDtypeStruct((E, C), jnp.float32),
    )(TJ, edge_attr, Wsc, We, om2, bn1, Wn2, bn2, Wn3, bn3)


def _sc_scatter(dst3d, msg, init):
    E, C = msg.shape
    N = init.shape[0] // NC
    EPW = E // NW
    NCHK = EPW // CH
    RPT = 640
    TAIL = N - RPT * (NS - 1)
    mesh = plsc.VectorSubcoreMesh(
        core_axis_name="c", subcore_axis_name="s", num_cores=NC, num_subcores=NS
    )

    @functools.partial(
        pl.kernel,
        mesh=mesh,
        compiler_params=pltpu.CompilerParams(needs_layout_passes=False),
        out_type=jax.ShapeDtypeStruct((NC * N, C), jnp.float32),
        scratch_types=[
            pltpu.VMEM((NCHK, CH), jnp.int32),
            pltpu.VMEM((CH, C), jnp.float32),
            pltpu.VMEM((CH, C), jnp.float32),
            pltpu.VMEM_SHARED((N, C), jnp.float32),
            pltpu.SemaphoreType.DMA,
            pltpu.SemaphoreType.DMA,
        ],
    )
    def k(dst3d_hbm, msg_hbm, z_hbm, agg_out, idx_v, mbuf0, mbuf1, agg_sh,
          sem0, sem1):
        c = lax.axis_index("c")
        s = lax.axis_index("s")
        w = c * NS + s
        cidx = pltpu.async_copy(dst3d_hbm.at[w], idx_v, sem0)

        @pl.when(s < NS - 1)
        def _():
            r0 = pl.multiple_of(s * RPT, 8)
            i0 = pl.multiple_of(c * N + s * RPT, 8)
            pltpu.sync_copy(z_hbm.at[pl.ds(i0, RPT)], agg_sh.at[pl.ds(r0, RPT)])

        @pl.when(s == NS - 1)
        def _():
            r0 = RPT * (NS - 1)
            i0 = pl.multiple_of(c * N + r0, 8)
            pltpu.sync_copy(z_hbm.at[pl.ds(i0, TAIL)], agg_sh.at[pl.ds(r0, TAIL)])

        cidx.wait()
        plsc.subcore_barrier()
        bufs = (mbuf0, mbuf1)
        sems = (sem0, sem1)

        def start(j, b):
            e0 = pl.multiple_of(w * EPW + j * CH, 8)
            pltpu.async_copy(msg_hbm.at[pl.ds(e0, CH)], bufs[b], sems[b])

        def finish(j, b):
            e0 = pl.multiple_of(w * EPW + j * CH, 8)
            pltpu.make_async_copy(msg_hbm.at[pl.ds(e0, CH)], bufs[b],
                                  sems[b]).wait()
            pltpu.sync_copy(bufs[b], agg_sh.at[idx_v.at[j]], add=True)

        start(0, 0)
        start(1, 1)

        @pl.loop(0, NCHK - 1, step=2)
        def _(j):
            finish(j, 0)

            @pl.when(j + 2 < NCHK)
            def _():
                start(j + 2, 0)

            finish(j + 1, 1)

            @pl.when(j + 3 < NCHK)
            def _():
                start(j + 3, 1)

        if NCHK % 2 == 1:
            finish(NCHK - 1, 0)
        plsc.subcore_barrier()

        @pl.when(s < NS - 1)
        def _():
            r0 = pl.multiple_of(s * RPT, 8)
            o0 = pl.multiple_of(c * N + s * RPT, 8)
            pltpu.sync_copy(agg_sh.at[pl.ds(r0, RPT)], agg_out.at[pl.ds(o0, RPT)])

        @pl.when(s == NS - 1)
        def _():
            r0 = RPT * (NS - 1)
            o0 = pl.multiple_of(c * N + r0, 8)
            pltpu.sync_copy(agg_sh.at[pl.ds(r0, TAIL)], agg_out.at[pl.ds(o0, TAIL)])

    return k(dst3d, msg, init)


def _final_stage(h, aggps, W_a2, b_a2, W_a3, b_a3):
    N, C = h.shape
    BN = 1000
    grid = (N // BN,)

    nparts = len(aggps)

    def body(*refs):
        h_ref = refs[0]
        aggs = refs[1:1 + 2 * nparts]
        wa2_ref, ba2_ref, wa3_ref, ba3_ref, out_ref = refs[1 + 2 * nparts:]
        hb = h_ref[...]
        for a in aggs:
            hb = hb + a[...]
        t = jnp.maximum(hb @ wa2_ref[...] + ba2_ref[...], 0.0)
        out_ref[...] = t @ wa3_ref[...] + ba3_ref[...]

    nb = N // BN
    agg_specs = []
    agg_args = []
    for a in aggps:
        agg_specs.append(pl.BlockSpec((BN, C), lambda i: (i, 0)))
        agg_specs.append(pl.BlockSpec((BN, C), lambda i: (i + nb, 0)))
        agg_args.extend([a, a])
    return pl.pallas_call(
        body,
        grid=grid,
        in_specs=[pl.BlockSpec((BN, C), lambda i: (i, 0))] + agg_specs + [
            pl.BlockSpec((C, C), lambda i: (0, 0)),
            pl.BlockSpec((1, C), lambda i: (0, 0)),
            pl.BlockSpec((C, C), lambda i: (0, 0)),
            pl.BlockSpec((1, C), lambda i: (0, 0)),
        ],
        out_specs=pl.BlockSpec((BN, C), lambda i: (i, 0)),
        out_shape=jax.ShapeDtypeStruct((N, C), jnp.float32),
    )(h, *agg_args, W_a2, b_a2, W_a3, b_a3)


def kernel(x, edge_index, edge_attr, x_pos, W_a1, b_a1, Wn1, bn1, Wn2, bn2,
           Wn3, bn3, W_a2, b_a2, W_a3, b_a3):
    N, C = x.shape
    E = edge_index.shape[1]
    NF = 16
    n_channels = 128
    omeg = jnp.asarray(
        [10.0 * (float(n_channels) ** (1.0 - 2.0 * i / NF)) for i in range(NF)],
        jnp.float32).reshape(1, NF)

    src = edge_index[0]
    dst = edge_index[1]
    rs2 = np.float32(0.7071067811865476)
    Wsc = Wn1[: 2 * NF] * rs2
    We = Wn1[2 * NF: 2 * NF + C] * rs2
    Wg = Wn1[2 * NF + C:] * rs2
    bn1 = bn1 * rs2
    Wn2 = Wn2 * np.float32(0.5)
    bn2 = bn2 * rs2
    Wn3 = Wn3 * rs2

    h, T = _node_stage(x, W_a1, b_a1.reshape(1, C), Wg)
    px, py, pz = x_pos[:, 0], x_pos[:, 1], x_pos[:, 2]
    om2 = omeg * omeg
    zer = jnp.zeros((NC * N, C), jnp.float32)
    G = NW * CH
    nblk = E // G
    if nblk >= 8:
        b1 = (nblk * 35) // 100
        b2 = (nblk * 37) // 100
        blks = [b1, b2, nblk - b1 - b2]
    else:
        blks = [nblk]
    bn1r, bn2r, bn3r = bn1.reshape(1, -1), bn2.reshape(1, -1), bn3.reshape(1, C)
    agg = zer
    e0 = 0
    for nb_i in blks:
        ei = nb_i * G
        srcP, dstP = src[e0:e0 + ei], dst[e0:e0 + ei]
        TJP = _sc_gather(srcP, dstP, T, px, py, pz)
        msgP = _edge_stage(TJP, edge_attr, Wsc, We, om2, bn1r, Wn2, bn2r,
                           Wn3, bn3r, blk_off=e0 // G)
        agg = _sc_scatter(dstP.reshape(NW, ei // G, CH), msgP, agg)
        e0 += ei
    return _final_stage(h, [agg], W_a2, b_a2.reshape(1, C), W_a3,
                        b_a3.reshape(1, C))


# speedup vs baseline: 6.6165x; 1.0183x over previous
import functools

import jax
import jax.numpy as jnp
import numpy as np
from jax import lax
from jax.experimental import pallas as pl
from jax.experimental.pallas import tpu as pltpu
from jax.experimental.pallas import tpu_sc as plsc

NC = 2
NS = 16
NW = NC * NS
CH = 80


def _sincos(ang):
    nf = jnp.round(ang * np.float32(2.0 / np.pi))
    ni = nf.astype(jnp.int32)
    x = ((ang - nf * np.float32(1.5703125))
         - nf * np.float32(4.837512969970703125e-4)) \
        - nf * np.float32(7.54978995489188216e-8)
    z = x * x
    sp = ((np.float32(-1.9515295891e-4) * z + np.float32(8.3321608736e-3)) * z
          + np.float32(-1.6666654611e-1)) * z * x + x
    cp = ((np.float32(2.443315711809948e-5) * z
           + np.float32(-1.388731625493765e-3)) * z
          + np.float32(4.166664568298827e-2)) * z * z \
        - np.float32(0.5) * z + np.float32(1.0)
    q0 = (ni & 1) != 0
    ssel = jnp.where(q0, cp, sp)
    csel = jnp.where(q0, sp, cp)
    sflip = (ni & 2) << 30
    cflip = ((ni + 1) & 2) << 30
    s = lax.bitcast_convert_type(
        lax.bitcast_convert_type(ssel, jnp.int32) ^ sflip, jnp.float32)
    c = lax.bitcast_convert_type(
        lax.bitcast_convert_type(csel, jnp.int32) ^ cflip, jnp.float32)
    return s, c


def _node_stage(x, W_a1, b_a1, Wg):
    N, C = x.shape
    BN = 1000
    grid = (N // BN,)

    def body(x_ref, wa1_ref, ba1_ref, wg_ref, h_ref, t_ref):
        hb = x_ref[...] @ wa1_ref[...] + ba1_ref[...]
        h_ref[...] = hb
        g = hb @ wg_ref[...]
        t_ref[...] = jnp.concatenate([g, jnp.zeros_like(g)], axis=1)

    return pl.pallas_call(
        body,
        grid=grid,
        in_specs=[
            pl.BlockSpec((BN, C), lambda i: (i, 0)),
            pl.BlockSpec((C, C), lambda i: (0, 0)),
            pl.BlockSpec((1, C), lambda i: (0, 0)),
            pl.BlockSpec((C, 64), lambda i: (0, 0)),
        ],
        out_specs=[
            pl.BlockSpec((BN, C), lambda i: (i, 0)),
            pl.BlockSpec((BN, C), lambda i: (i, 0)),
        ],
        out_shape=[
            jax.ShapeDtypeStruct((N, C), jnp.float32),
            jax.ShapeDtypeStruct((N, C), jnp.float32),
        ],
    )(x, W_a1, b_a1, Wg)


def _sc_gather(src, dst, T, px, py, pz):
    E = src.shape[0]
    N, C = T.shape
    EPW = E // NW
    NCHK = EPW // CH
    mesh = plsc.VectorSubcoreMesh(
        core_axis_name="c", subcore_axis_name="s", num_cores=NC, num_subcores=NS
    )

    @functools.partial(
        pl.kernel,
        mesh=mesh,
        out_type=jax.ShapeDtypeStruct((E, C), jnp.float32),
        compiler_params=pltpu.CompilerParams(needs_layout_passes=False),
        scratch_types=[
            pltpu.VMEM((EPW,), jnp.int32),
            pltpu.VMEM((EPW,), jnp.int32),
            pltpu.VMEM((N,), jnp.float32),
            pltpu.VMEM((N,), jnp.float32),
            pltpu.VMEM((N,), jnp.float32),
            pltpu.VMEM((CH, C), jnp.float32),
            pltpu.VMEM((CH, C), jnp.float32),
            pltpu.SemaphoreType.DMA,
            pltpu.SemaphoreType.DMA,
        ],
    )
    def k(src_hbm, dst_hbm, t_hbm, px_hbm, py_hbm, pz_hbm, tj_out,
          src_v, dst_v, px_v, py_v, pz_v, tjbuf0, tjbuf1, sem0, sem1):
        w = lax.axis_index("c") * NS + lax.axis_index("s")
        base = pl.multiple_of(w * EPW, 8)
        c1 = pltpu.async_copy(src_hbm.at[pl.ds(base, EPW)], src_v, sem0)
        c2 = pltpu.async_copy(dst_hbm.at[pl.ds(base, EPW)], dst_v, sem0)
        c3 = pltpu.async_copy(px_hbm, px_v, sem1)
        c4 = pltpu.async_copy(py_hbm, py_v, sem1)
        c5 = pltpu.async_copy(pz_hbm, pz_v, sem1)
        c1.wait()
        c2.wait()
        c3.wait()
        c4.wait()
        c5.wait()
        lane = lax.iota(jnp.int32, 16)
        bufs = (tjbuf0, tjbuf1)
        sems = (sem0, sem1)

        def start(j, b):
            ch0 = pl.multiple_of(j * CH, 8)
            pltpu.async_copy(t_hbm.at[src_v.at[pl.ds(ch0, CH)]], bufs[b],
                             sems[b])

        def finish(j, b):
            ch0 = pl.multiple_of(j * CH, 8)
            pltpu.make_async_copy(t_hbm.at[src_v.at[pl.ds(ch0, CH)]], bufs[b],
                                  sems[b]).wait()
            buf = bufs[b]
            for gi in range(CH // 16):
                off = pl.multiple_of(j * CH + gi * 16, 8)
                s16 = src_v[pl.ds(off, 16)]
                d16 = dst_v[pl.ds(off, 16)]
                dx = plsc.load_gather(px_v, [d16]) - plsc.load_gather(px_v, [s16])
                dy = plsc.load_gather(py_v, [d16]) - plsc.load_gather(py_v, [s16])
                dz = plsc.load_gather(pz_v, [d16]) - plsc.load_gather(pz_v, [s16])
                r2 = dx * dx + dy * dy + dz * dz
                for cix in range(16):
                    plsc.store_scatter(
                        buf, [gi * 16 + lane, jnp.full((16,), 64 + cix,
                                                       jnp.int32)], r2)
            pltpu.sync_copy(
                buf, tj_out.at[pl.ds(pl.multiple_of(base + j * CH, 8), CH)])

        start(0, 0)
        start(1, 1)

        @pl.loop(0, NCHK - 1, step=2)
        def _(j):
            finish(j, 0)

            @pl.when(j + 2 < NCHK)
            def _():
                start(j + 2, 0)

            finish(j + 1, 1)

            @pl.when(j + 3 < NCHK)
            def _():
                start(j + 3, 1)

        if NCHK % 2 == 1:
            finish(NCHK - 1, 0)

    return k(src, dst, T, px, py, pz)


def _edge_stage(TJ, edge_attr, Wsc, We, om2, bn1, Wn2, bn2, Wn3, bn3,
                blk_off=0):
    E = TJ.shape[0]
    C = edge_attr.shape[1]
    MID = Wn2.shape[0]
    BE = 2560
    grid = (E // BE,)

    def body(tj_ref, ea_ref, wsc_ref, we_ref, om_ref, bn1_ref,
             w2_ref, bn2_ref, w3_ref, bn3_ref, out_ref):
        tj = tj_ref[...]
        g = tj[:, :64]
        ang = jnp.sqrt(tj[:, 64:80] * om_ref[...])
        sn, cs = _sincos(ang)
        sc = jnp.concatenate([sn, cs], axis=1)
        p1 = (sc @ wsc_ref[...]
              + ea_ref[...] @ we_ref[...] + g + bn1_ref[...])
        q1 = p1 * lax.erf(p1) + p1
        p2 = q1 @ w2_ref[...] + bn2_ref[...]
        q2 = p2 * lax.erf(p2) + p2
        out_ref[...] = q2 @ w3_ref[...] + bn3_ref[...]

    return pl.pallas_call(
        body,
        grid=grid,
        in_specs=[
            pl.BlockSpec((BE, C), lambda i: (i, 0)),
            pl.BlockSpec((BE, C), lambda i: (i + blk_off, 0)),
            pl.BlockSpec((32, MID), lambda i: (0, 0)),
            pl.BlockSpec((C, MID), lambda i: (0, 0)),
            pl.BlockSpec((1, 16), lambda i: (0, 0)),
            pl.BlockSpec((1, MID), lambda i: (0, 0)),
            pl.BlockSpec((MID, MID), lambda i: (0, 0)),
            pl.BlockSpec((1, MID), lambda i: (0, 0)),
            pl.BlockSpec((MID, C), lambda i: (0, 0)),
            pl.BlockSpec((1, C), lambda i: (0, 0)),
        ],
        out_specs=pl.BlockSpec((BE, C), lambda i: (i, 0)),
        out_shape=jax.ShapeDtypeStruct((E, C), jnp.float32),
    )(TJ, edge_attr, Wsc, We, om2, bn1, Wn2, bn2, Wn3, bn3)


def _sc_scatter(dst3d, msg, init):
    E, C = msg.shape
    N = init.shape[0] // NC
    EPW = E // NW
    NCHK = EPW // CH
    RPT = 640
    TAIL = N - RPT * (NS - 1)
    mesh = plsc.VectorSubcoreMesh(
        core_axis_name="c", subcore_axis_name="s", num_cores=NC, num_subcores=NS
    )

    @functools.partial(
        pl.kernel,
        mesh=mesh,
        compiler_params=pltpu.CompilerParams(needs_layout_passes=False),
        out_type=jax.ShapeDtypeStruct((NC * N, C), jnp.float32),
        scratch_types=[
            pltpu.VMEM((NCHK, CH), jnp.int32),
            pltpu.VMEM((CH, C), jnp.float32),
            pltpu.VMEM((CH, C), jnp.float32),
            pltpu.VMEM_SHARED((N, C), jnp.float32),
            pltpu.SemaphoreType.DMA,
            pltpu.SemaphoreType.DMA,
        ],
    )
    def k(dst3d_hbm, msg_hbm, z_hbm, agg_out, idx_v, mbuf0, mbuf1, agg_sh,
          sem0, sem1):
        c = lax.axis_index("c")
        s = lax.axis_index("s")
        w = c * NS + s
        cidx = pltpu.async_copy(dst3d_hbm.at[w], idx_v, sem0)

        @pl.when(s < NS - 1)
        def _():
            r0 = pl.multiple_of(s * RPT, 8)
            i0 = pl.multiple_of(c * N + s * RPT, 8)
            pltpu.sync_copy(z_hbm.at[pl.ds(i0, RPT)], agg_sh.at[pl.ds(r0, RPT)])

        @pl.when(s == NS - 1)
        def _():
            r0 = RPT * (NS - 1)
            i0 = pl.multiple_of(c * N + r0, 8)
            pltpu.sync_copy(z_hbm.at[pl.ds(i0, TAIL)], agg_sh.at[pl.ds(r0, TAIL)])

        cidx.wait()
        plsc.subcore_barrier()
        bufs = (mbuf0, mbuf1)
        sems = (sem0, sem1)

        def start(j, b):
            e0 = pl.multiple_of(w * EPW + j * CH, 8)
            pltpu.async_copy(msg_hbm.at[pl.ds(e0, CH)], bufs[b], sems[b])

        def finish(j, b):
            e0 = pl.multiple_of(w * EPW + j * CH, 8)
            pltpu.make_async_copy(msg_hbm.at[pl.ds(e0, CH)], bufs[b],
                                  sems[b]).wait()
            pltpu.sync_copy(bufs[b], agg_sh.at[idx_v.at[j]], add=True)

        start(0, 0)
        start(1, 1)

        @pl.loop(0, NCHK - 1, step=2)
        def _(j):
            finish(j, 0)

            @pl.when(j + 2 < NCHK)
            def _():
                start(j + 2, 0)

            finish(j + 1, 1)

            @pl.when(j + 3 < NCHK)
            def _():
                start(j + 3, 1)

        if NCHK % 2 == 1:
            finish(NCHK - 1, 0)
        plsc.subcore_barrier()

        @pl.when(s < NS - 1)
        def _():
            r0 = pl.multiple_of(s * RPT, 8)
            o0 = pl.multiple_of(c * N + s * RPT, 8)
            pltpu.sync_copy(agg_sh.at[pl.ds(r0, RPT)], agg_out.at[pl.ds(o0, RPT)])

        @pl.when(s == NS - 1)
        def _():
            r0 = RPT * (NS - 1)
            o0 = pl.multiple_of(c * N + r0, 8)
            pltpu.sync_copy(agg_sh.at[pl.ds(r0, TAIL)], agg_out.at[pl.ds(o0, TAIL)])

    return k(dst3d, msg, init)


def _final_stage(h, aggps, W_a2, b_a2, W_a3, b_a3):
    N, C = h.shape
    BN = 1000
    grid = (N // BN,)

    nparts = len(aggps)

    def body(*refs):
        h_ref = refs[0]
        aggs = refs[1:1 + 2 * nparts]
        wa2_ref, ba2_ref, wa3_ref, ba3_ref, out_ref = refs[1 + 2 * nparts:]
        hb = h_ref[...]
        for a in aggs:
            hb = hb + a[...]
        t = jnp.maximum(hb @ wa2_ref[...] + ba2_ref[...], 0.0)
        out_ref[...] = t @ wa3_ref[...] + ba3_ref[...]

    nb = N // BN
    agg_specs = []
    agg_args = []
    for a in aggps:
        agg_specs.append(pl.BlockSpec((BN, C), lambda i: (i, 0)))
        agg_specs.append(pl.BlockSpec((BN, C), lambda i: (i + nb, 0)))
        agg_args.extend([a, a])
    return pl.pallas_call(
        body,
        grid=grid,
        in_specs=[pl.BlockSpec((BN, C), lambda i: (i, 0))] + agg_specs + [
            pl.BlockSpec((C, C), lambda i: (0, 0)),
            pl.BlockSpec((1, C), lambda i: (0, 0)),
            pl.BlockSpec((C, C), lambda i: (0, 0)),
            pl.BlockSpec((1, C), lambda i: (0, 0)),
        ],
        out_specs=pl.BlockSpec((BN, C), lambda i: (i, 0)),
        out_shape=jax.ShapeDtypeStruct((N, C), jnp.float32),
    )(h, *agg_args, W_a2, b_a2, W_a3, b_a3)


def kernel(x, edge_index, edge_attr, x_pos, W_a1, b_a1, Wn1, bn1, Wn2, bn2,
           Wn3, bn3, W_a2, b_a2, W_a3, b_a3):
    N, C = x.shape
    E = edge_index.shape[1]
    NF = 16
    n_channels = 128
    omeg = jnp.asarray(
        [10.0 * (float(n_channels) ** (1.0 - 2.0 * i / NF)) for i in range(NF)],
        jnp.float32).reshape(1, NF)

    src = edge_index[0]
    dst = edge_index[1]
    rs2 = np.float32(0.7071067811865476)
    Wsc = Wn1[: 2 * NF] * rs2
    We = Wn1[2 * NF: 2 * NF + C] * rs2
    Wg = Wn1[2 * NF + C:] * rs2
    bn1 = bn1 * rs2
    Wn2 = Wn2 * np.float32(0.5)
    bn2 = bn2 * rs2
    Wn3 = Wn3 * rs2

    h, T = _node_stage(x, W_a1, b_a1.reshape(1, C), Wg)
    px, py, pz = x_pos[:, 0], x_pos[:, 1], x_pos[:, 2]
    om2 = omeg * omeg
    zer = jnp.zeros((NC * N, C), jnp.float32)
    G = NW * CH
    nblk = E // G
    if nblk >= 8:
        b1 = (nblk * 26) // 100
        b2 = (nblk * 41) // 100
        blks = [b1, b2, nblk - b1 - b2]
    else:
        blks = [nblk]
    bn1r, bn2r, bn3r = bn1.reshape(1, -1), bn2.reshape(1, -1), bn3.reshape(1, C)
    agg = zer
    e0 = 0
    for nb_i in blks:
        ei = nb_i * G
        srcP, dstP = src[e0:e0 + ei], dst[e0:e0 + ei]
        TJP = _sc_gather(srcP, dstP, T, px, py, pz)
        msgP = _edge_stage(TJP, edge_attr, Wsc, We, om2, bn1r, Wn2, bn2r,
                           Wn3, bn3r, blk_off=e0 // G)
        agg = _sc_scatter(dstP.reshape(NW, ei // G, CH), msgP, agg)
        e0 += ei
    return _final_stage(h, [agg], W_a2, b_a2.reshape(1, C), W_a3,
                        b_a3.reshape(1, C))
